# baseline (speedup 1.0000x reference)
import functools

import jax
import jax.numpy as jnp
from jax import lax
from jax.experimental import pallas as pl
from jax.experimental.pallas import tpu as pltpu
from jax.experimental.pallas import tpu_sc as plsc

N_PROP = 20000
N_PAD = 20480
N_GT = 64
NW = 32
CHUNK = N_PAD // NW
NUM_CLASSES = 80
IOU_THRESH = 0.5
K_FG = 128
K_BG = 384
K_PAD = 400
NEG_INF = float("-inf")


def _iota16():
    return lax.iota(jnp.int32, 16)


def _phase_a_body(px1, py1, px2, py2, sc, gx1, gy1, gx2, gy2, gcl,
                  iou_out, fg_out, bg_out, cls_out,
                  px1_v, py1_v, px2_v, py2_v, sc_v,
                  g1_v, g2_v, g3_v, g4_v, gc_v,
                  iou_s, fg_s, bg_s, cls_s):
    wid = lax.axis_index("s") * 2 + lax.axis_index("c")
    base = wid * CHUNK

    pltpu.sync_copy(px1.at[pl.ds(base, CHUNK)], px1_v)
    pltpu.sync_copy(py1.at[pl.ds(base, CHUNK)], py1_v)
    pltpu.sync_copy(px2.at[pl.ds(base, CHUNK)], px2_v)
    pltpu.sync_copy(py2.at[pl.ds(base, CHUNK)], py2_v)
    pltpu.sync_copy(sc.at[pl.ds(base, CHUNK)], sc_v)
    pltpu.sync_copy(gx1, g1_v)
    pltpu.sync_copy(gy1, g2_v)
    pltpu.sync_copy(gx2, g3_v)
    pltpu.sync_copy(gy2, g4_v)
    pltpu.sync_copy(gcl, gc_v)

    def chunk_body(v, _):
        o = v * 16
        p_x1 = px1_v[pl.ds(o, 16)]
        p_y1 = py1_v[pl.ds(o, 16)]
        p_x2 = px2_v[pl.ds(o, 16)]
        p_y2 = py2_v[pl.ds(o, 16)]
        s = sc_v[pl.ds(o, 16)]
        ap = (p_x2 - p_x1) * (p_y2 - p_y1)

        def g_body(g, c):
            bi, bc = c
            go = g * 16
            g_x1 = g1_v[pl.ds(go, 16)]
            g_y1 = g2_v[pl.ds(go, 16)]
            g_x2 = g3_v[pl.ds(go, 16)]
            g_y2 = g4_v[pl.ds(go, 16)]
            g_c = gc_v[pl.ds(go, 16)]
            ag = (g_x2 - g_x1) * (g_y2 - g_y1)
            w = jnp.maximum(jnp.minimum(g_x2, p_x2) - jnp.maximum(g_x1, p_x1), 0.0)
            h = jnp.maximum(jnp.minimum(g_y2, p_y2) - jnp.maximum(g_y1, p_y1), 0.0)
            inter = w * h
            union = ag + ap - inter
            iou = inter / jnp.maximum(union, 1e-9)
            upd = iou > bi
            return (jnp.where(upd, iou, bi), jnp.where(upd, g_c, bc))

        bi0 = jnp.full((16,), -1.0, jnp.float32)
        bc0 = jnp.zeros((16,), jnp.int32)
        bi, bc = lax.fori_loop(0, N_GT, g_body, (bi0, bc0))

        gidx = base + o + _iota16()
        valid = gidx < N_PROP
        matched = bi >= IOU_THRESH
        neg = jnp.full((16,), NEG_INF, jnp.float32)
        fg = jnp.where(
            valid,
            jnp.where(matched, bi, jnp.full((16,), -1.0, jnp.float32)), neg)
        bg = jnp.where(
            valid,
            jnp.where(matched, jnp.full((16,), -1e9, jnp.float32), s), neg)
        cl = jnp.where(matched, bc, jnp.full((16,), NUM_CLASSES, jnp.int32))

        iou_s[pl.ds(o, 16)] = bi
        fg_s[pl.ds(o, 16)] = fg
        bg_s[pl.ds(o, 16)] = bg
        cls_s[pl.ds(o, 16)] = cl
        return ()

    lax.fori_loop(0, CHUNK // 16, chunk_body, ())

    pltpu.sync_copy(iou_s, iou_out.at[pl.ds(base, CHUNK)])
    pltpu.sync_copy(fg_s, fg_out.at[pl.ds(base, CHUNK)])
    pltpu.sync_copy(bg_s, bg_out.at[pl.ds(base, CHUNK)])
    pltpu.sync_copy(cls_s, cls_out.at[pl.ds(base, CHUNK)])


def _topk_tile(key_hbm, cls_hbm, sv_out, si_out, scl_out, k, out_off,
               key_v, u_v, cls_v, hist, candU, candV, candI, candC,
               outV, outI, outC):
    nv = N_PAD // 16
    pltpu.sync_copy(key_hbm, key_v)
    pltpu.sync_copy(cls_hbm, cls_v)

    def u_body(v, _):
        o = v * 16
        b = plsc.bitcast(key_v[pl.ds(o, 16)], jnp.int32)
        u = jnp.where(b < 0, ~b, b ^ jnp.int32(-2147483648))
        u_v[pl.ds(o, 16)] = plsc.bitcast(u, jnp.uint32)
        return ()

    lax.fori_loop(0, nv, u_body, ())

    one = jnp.ones((16,), jnp.int32)
    zero = jnp.zeros((16,), jnp.int32)
    iot = _iota16()

    pre = jnp.uint32(0)
    rem = jnp.int32(k)
    m = jnp.int32(0)
    for shift, width in ((21, 11), (10, 11), (0, 10)):
        nbins = 1 << width
        nb = nbins // 16

        def z_body(v, _):
            hist[pl.ds(v * 16, 16)] = zero
            return ()

        lax.fori_loop(0, nb, z_body, ())

        dmask = jnp.uint32(nbins - 1)
        hi = shift + width
        pre_hi = lax.shift_right_logical(pre, jnp.uint32(hi)) if hi < 32 else None

        def acc_body(v, _, shift=shift, hi=hi, dmask=dmask, pre_hi=pre_hi):
            u = u_v[pl.ds(v * 16, 16)]
            dig = lax.shift_right_logical(u, jnp.uint32(shift)) & dmask
            if pre_hi is None:
                inc = one
            else:
                uh = lax.shift_right_logical(u, jnp.uint32(hi))
                inc = jnp.where(uh == jnp.broadcast_to(pre_hi, (16,)), one, zero)
            plsc.addupdate_scatter(hist, [plsc.bitcast(dig, jnp.int32)], inc)
            return ()

        lax.fori_loop(0, nv, acc_body, ())

        def s_body(j, c, nb=nb):
            above, found, d, g = c
            v = nb - 1 - j
            h = hist[pl.ds(v * 16, 16)]
            s_ge = lax.rev(plsc.cumsum(lax.rev(h, (0,))), (0,))
            tot = above + s_ge
            npos = jnp.max(
                jnp.where(tot >= jnp.broadcast_to(rem, (16,)), one, zero)
                * (iot + 1))
            sel = jnp.where(found == 0, npos, 0)
            lv = jnp.broadcast_to(npos - 1, (16,))
            gv = above + jnp.sum(jnp.where(iot > lv, h, zero))
            dv = v * 16 + npos - 1
            d = jnp.where(sel > 0, dv, d)
            g = jnp.where(sel > 0, gv, g)
            found = jnp.maximum(found, jnp.where(npos > 0, 1, 0))
            above = above + jnp.sum(h)
            return (above, found, d, g)

        _, _, d, g = lax.fori_loop(
            0, nb, s_body,
            (jnp.int32(0), jnp.int32(0), jnp.int32(0), jnp.int32(0)))
        pre = pre | lax.shift_left(
            lax.bitcast_convert_type(d, jnp.uint32), jnp.uint32(shift))
        rem = rem - g
        m = m + g

    t_u = pre

    def pad_body(v, _):
        o = v * 16
        candU[pl.ds(o, 16)] = jnp.full((16,), jnp.int32(-2147483648))
        candI[pl.ds(o, 16)] = jnp.full((16,), jnp.int32(2147483647))
        candV[pl.ds(o, 16)] = jnp.zeros((16,), jnp.float32)
        candC[pl.ds(o, 16)] = jnp.zeros((16,), jnp.int32)
        return ()

    lax.fori_loop(0, K_PAD // 16, pad_body, ())

    tv = jnp.broadcast_to(t_u, (16,))
    sgn = jnp.int32(-2147483648)

    def comp_body(v, carry):
        off, eqc = carry
        o = v * 16
        u = u_v[pl.ds(o, 16)]
        any_rel = jnp.sum(jnp.where(u >= tv, one, zero))

        def do(carry):
            off, eqc = carry
            kv = key_v[pl.ds(o, 16)]
            cv = cls_v[pl.ds(o, 16)]
            gidx = o + _iota16()
            m_gt = u > tv
            s = plsc.bitcast(u, jnp.int32) ^ sgn
            plsc.store_compressed(candU.at[pl.ds(off, 16)], s, mask=m_gt)
            plsc.store_compressed(candV.at[pl.ds(off, 16)], kv, mask=m_gt)
            plsc.store_compressed(candI.at[pl.ds(off, 16)], gidx, mask=m_gt)
            plsc.store_compressed(candC.at[pl.ds(off, 16)], cv, mask=m_gt)
            n_gt = jnp.sum(jnp.where(m_gt, one, zero))
            m_eq = u == tv
            eq1 = jnp.where(m_eq, one, zero)
            pos = eqc + plsc.cumsum(eq1) - 1
            dest = m + pos
            wmask = jnp.where(m_eq, dest, jnp.full((16,), k, jnp.int32)) < k
            destc = jnp.clip(dest, 0, k - 1)
            plsc.store_scatter(outV, [destc], kv, mask=wmask)
            plsc.store_scatter(outI, [destc], gidx, mask=wmask)
            plsc.store_scatter(outC, [destc], cv, mask=wmask)
            return off + n_gt, eqc + jnp.sum(eq1)

        return lax.cond(any_rel > 0, do, lambda c: c, (off, eqc))

    lax.fori_loop(0, nv, comp_body, (jnp.int32(0), jnp.int32(0)))

    na = k // 16

    def a_body(a, _):
        ao = a * 16
        aU = candU[pl.ds(ao, 16)]
        aI = candI[pl.ds(ao, 16)]
        aV = candV[pl.ds(ao, 16)]
        aC = candC[pl.ds(ao, 16)]

        def b_body(bb, acc):
            def r_body(r, acc):
                idx = bb * 16 + ((_iota16() + r) & 15)
                bU = plsc.load_gather(candU, [idx])
                bI = plsc.load_gather(candI, [idx])
                tie = jnp.where(bI < aI, one, zero)
                better = jnp.where(bU > aU, one,
                                   jnp.where(bU == aU, tie, zero))
                return acc + better

            return lax.fori_loop(0, 16, r_body, acc)

        rank = lax.fori_loop(0, na + 1, b_body, zero)
        lanepos = ao + _iota16()
        wmask = jnp.where(lanepos < m, rank, jnp.full((16,), k, jnp.int32)) < k
        rc = jnp.clip(rank, 0, k - 1)
        plsc.store_scatter(outV, [rc], aV, mask=wmask)
        plsc.store_scatter(outI, [rc], aI, mask=wmask)
        plsc.store_scatter(outC, [rc], aC, mask=wmask)
        return ()

    lax.fori_loop(0, na, a_body, ())

    pltpu.sync_copy(outV.at[pl.ds(0, k)], sv_out.at[pl.ds(out_off, k)])
    pltpu.sync_copy(outI.at[pl.ds(0, k)], si_out.at[pl.ds(out_off, k)])
    pltpu.sync_copy(outC.at[pl.ds(0, k)], scl_out.at[pl.ds(out_off, k)])


def _phase_b_body(fg_hbm, bg_hbm, cls_hbm, sv_out, si_out, scl_out,
                  key_v, u_v, cls_v, hist, candU, candV, candI, candC,
                  outV, outI, outC):
    nc = lax.axis_size("c")
    wid = lax.axis_index("s") * nc + lax.axis_index("c")

    @pl.when(wid == 0)
    def _fg():
        _topk_tile(fg_hbm, cls_hbm, sv_out, si_out, scl_out, K_FG, 0,
                   key_v, u_v, cls_v, hist, candU, candV, candI, candC,
                   outV, outI, outC)

    @pl.when(wid == 1)
    def _bg():
        _topk_tile(bg_hbm, cls_hbm, sv_out, si_out, scl_out, K_BG, K_FG,
                   key_v, u_v, cls_v, hist, candU, candV, candI, candC,
                   outV, outI, outC)


@jax.jit
def kernel(proposal_boxes, gt_boxes, scores, gt_classes):
    mesh = plsc.VectorSubcoreMesh(core_axis_name="c", subcore_axis_name="s",
                                  num_cores=2, num_subcores=16)
    f32 = jnp.float32
    i32 = jnp.int32

    pb = jnp.pad(proposal_boxes, ((0, N_PAD - N_PROP), (0, 0)))
    px1, py1, px2, py2 = (pb[:, i] for i in range(4))
    sc = jnp.pad(scores, (0, N_PAD - N_PROP))
    g_rep = [jnp.repeat(gt_boxes[:, i], 16) for i in range(4)]
    gcl_rep = jnp.repeat(gt_classes, 16)

    phase_a = pl.kernel(
        _phase_a_body,
        out_type=(
            jax.ShapeDtypeStruct((N_PAD,), f32),
            jax.ShapeDtypeStruct((N_PAD,), f32),
            jax.ShapeDtypeStruct((N_PAD,), f32),
            jax.ShapeDtypeStruct((N_PAD,), i32),
        ),
        mesh=mesh,
        compiler_params=pltpu.CompilerParams(needs_layout_passes=False),
        scratch_types=[
            pltpu.VMEM((CHUNK,), f32), pltpu.VMEM((CHUNK,), f32),
            pltpu.VMEM((CHUNK,), f32), pltpu.VMEM((CHUNK,), f32),
            pltpu.VMEM((CHUNK,), f32),
            pltpu.VMEM((N_GT * 16,), f32), pltpu.VMEM((N_GT * 16,), f32),
            pltpu.VMEM((N_GT * 16,), f32), pltpu.VMEM((N_GT * 16,), f32),
            pltpu.VMEM((N_GT * 16,), i32),
            pltpu.VMEM((CHUNK,), f32), pltpu.VMEM((CHUNK,), f32),
            pltpu.VMEM((CHUNK,), f32), pltpu.VMEM((CHUNK,), i32),
        ],
    )
    iou_p, fg_key, bg_key, cls = phase_a(
        px1, py1, px2, py2, sc, g_rep[0], g_rep[1], g_rep[2], g_rep[3], gcl_rep)

    phase_b = pl.kernel(
        _phase_b_body,
        out_type=(
            jax.ShapeDtypeStruct((512,), f32),
            jax.ShapeDtypeStruct((512,), i32),
            jax.ShapeDtypeStruct((512,), i32),
        ),
        mesh=mesh,
        compiler_params=pltpu.CompilerParams(needs_layout_passes=False),
        scratch_types=[
            pltpu.VMEM((N_PAD,), f32),
            pltpu.VMEM((N_PAD,), jnp.uint32),
            pltpu.VMEM((N_PAD,), i32),
            pltpu.VMEM((2048,), i32),
            pltpu.VMEM((K_PAD,), i32), pltpu.VMEM((K_PAD,), f32),
            pltpu.VMEM((K_PAD,), i32), pltpu.VMEM((K_PAD,), i32),
            pltpu.VMEM((K_BG,), f32), pltpu.VMEM((K_BG,), i32),
            pltpu.VMEM((K_BG,), i32),
        ],
    )
    sv, si, scl = phase_b(fg_key, bg_key, cls)

    return iou_p[:N_PROP], si, scl, sv


# speedup vs baseline: 2.1096x; 2.1096x over previous
import functools

import jax
import jax.numpy as jnp
from jax import lax
from jax.experimental import pallas as pl
from jax.experimental.pallas import tpu as pltpu
from jax.experimental.pallas import tpu_sc as plsc

N_PROP = 20000
N_PAD = 20480
N_GT = 64
NW = 32
CHUNK = N_PAD // NW
NUM_CLASSES = 80
IOU_THRESH = 0.5
K_FG = 128
K_BG = 384
K_PAD = 400
OUT_PAD = 448
NEG_INF = float("-inf")


def _iota16():
    return lax.iota(jnp.int32, 16)


def _phase_a_body(px1, py1, px2, py2, sc, gx1, gy1, gx2, gy2, gcl,
                  iou_out, fg_out, bg_out, cls_out,
                  px1_v, py1_v, px2_v, py2_v, sc_v,
                  g1_v, g2_v, g3_v, g4_v, gc_v,
                  iou_s, fg_s, bg_s, cls_s):
    wid = lax.axis_index("s") * 2 + lax.axis_index("c")
    base = wid * CHUNK

    pltpu.sync_copy(px1.at[pl.ds(base, CHUNK)], px1_v)
    pltpu.sync_copy(py1.at[pl.ds(base, CHUNK)], py1_v)
    pltpu.sync_copy(px2.at[pl.ds(base, CHUNK)], px2_v)
    pltpu.sync_copy(py2.at[pl.ds(base, CHUNK)], py2_v)
    pltpu.sync_copy(sc.at[pl.ds(base, CHUNK)], sc_v)
    pltpu.sync_copy(gx1, g1_v)
    pltpu.sync_copy(gy1, g2_v)
    pltpu.sync_copy(gx2, g3_v)
    pltpu.sync_copy(gy2, g4_v)
    pltpu.sync_copy(gcl, gc_v)

    def chunk_body(v, _):
        o = v * 16
        p_x1 = px1_v[pl.ds(o, 16)]
        p_y1 = py1_v[pl.ds(o, 16)]
        p_x2 = px2_v[pl.ds(o, 16)]
        p_y2 = py2_v[pl.ds(o, 16)]
        s = sc_v[pl.ds(o, 16)]
        ap = (p_x2 - p_x1) * (p_y2 - p_y1)

        def g_body(g, c):
            bi, bc = c
            go = g * 16
            g_x1 = g1_v[pl.ds(go, 16)]
            g_y1 = g2_v[pl.ds(go, 16)]
            g_x2 = g3_v[pl.ds(go, 16)]
            g_y2 = g4_v[pl.ds(go, 16)]
            g_c = gc_v[pl.ds(go, 16)]
            ag = (g_x2 - g_x1) * (g_y2 - g_y1)
            w = jnp.maximum(jnp.minimum(g_x2, p_x2) - jnp.maximum(g_x1, p_x1), 0.0)
            h = jnp.maximum(jnp.minimum(g_y2, p_y2) - jnp.maximum(g_y1, p_y1), 0.0)
            inter = w * h
            union = ag + ap - inter
            iou = inter / jnp.maximum(union, 1e-9)
            upd = iou > bi
            return (jnp.where(upd, iou, bi), jnp.where(upd, g_c, bc))

        bi0 = jnp.full((16,), -1.0, jnp.float32)
        bc0 = jnp.zeros((16,), jnp.int32)
        bi, bc = lax.fori_loop(0, N_GT, g_body, (bi0, bc0))

        gidx = base + o + _iota16()
        valid = gidx < N_PROP
        matched = bi >= IOU_THRESH
        neg = jnp.full((16,), NEG_INF, jnp.float32)
        fg = jnp.where(
            valid,
            jnp.where(matched, bi, jnp.full((16,), -1.0, jnp.float32)), neg)
        bg = jnp.where(
            valid,
            jnp.where(matched, jnp.full((16,), -1e9, jnp.float32), s), neg)
        cl = jnp.where(matched, bc, jnp.full((16,), NUM_CLASSES, jnp.int32))

        iou_s[pl.ds(o, 16)] = bi
        fg_s[pl.ds(o, 16)] = fg
        bg_s[pl.ds(o, 16)] = bg
        cls_s[pl.ds(o, 16)] = cl
        return ()

    lax.fori_loop(0, CHUNK // 16, chunk_body, ())

    pltpu.sync_copy(iou_s, iou_out.at[pl.ds(base, CHUNK)])
    pltpu.sync_copy(fg_s, fg_out.at[pl.ds(base, CHUNK)])
    pltpu.sync_copy(bg_s, bg_out.at[pl.ds(base, CHUNK)])
    pltpu.sync_copy(cls_s, cls_out.at[pl.ds(base, CHUNK)])


SLICE = N_PAD // 16
NVB = SLICE // 16
INT_MIN = -2147483648
INT_MAX = 2147483647


def _phase_b_body(fg_hbm, bg_hbm, cls_hbm, sv_out, si_out, scl_out,
                  key_v, u_v, cls_v, hist, hidx,
                  candU, candI, candC, candD,
                  tieV, tieI, tieC, tieD,
                  dU, dI, dC, st16a, st16b, st16c, st16d, cnt16, cntb,
                  sh_hist, sh_cnt, sh_candU, sh_candI, sh_candC,
                  sh_outV, sh_outI, sh_outC):
    cid = lax.axis_index("c")
    tid = lax.axis_index("s")
    k = jnp.where(cid == 0, K_FG, K_BG)
    kvec = jnp.broadcast_to(k, (16,))
    one = jnp.ones((16,), jnp.int32)
    zero = jnp.zeros((16,), jnp.int32)
    iot = _iota16()
    sgn = jnp.int32(INT_MIN)
    base = tid * SLICE

    @pl.when(cid == 0)
    def _():
        pltpu.sync_copy(fg_hbm.at[pl.ds(base, SLICE)], key_v)

    @pl.when(cid == 1)
    def _():
        pltpu.sync_copy(bg_hbm.at[pl.ds(base, SLICE)], key_v)

    pltpu.sync_copy(cls_hbm.at[pl.ds(base, SLICE)], cls_v)

    def u_body(v, _):
        o = v * 16
        b = plsc.bitcast(key_v[pl.ds(o, 16)], jnp.int32)
        u = jnp.where(b < 0, ~b, b ^ sgn)
        u_v[pl.ds(o, 16)] = plsc.bitcast(u, jnp.uint32)
        return ()

    lax.fori_loop(0, NVB, u_body, ())

    hbase = cid * 2048

    def hx_body(v, _):
        hidx[pl.ds(v * 16, 16)] = hbase + v * 16 + iot
        return ()

    lax.fori_loop(0, 128, hx_body, ())

    pre = jnp.uint32(0)
    rem = k
    for shift, width in ((21, 11), (10, 11), (0, 10)):
        nbins = 1 << width
        nb = nbins // 16

        def z_body(v, _):
            hist[pl.ds(v * 16, 16)] = zero
            return ()

        lax.fori_loop(0, 128, z_body, ())
        plsc.subcore_barrier()

        @pl.when(tid == 0)
        def _():
            pltpu.sync_copy(hist.at[pl.ds(0, 2048)],
                            sh_hist.at[pl.ds(hbase, 2048)])

        plsc.subcore_barrier()

        dmask = jnp.uint32(nbins - 1)
        hi = shift + width
        pre_hi = lax.shift_right_logical(pre, jnp.uint32(hi)) if hi < 32 else None

        def acc_body(v, _, shift=shift, hi=hi, dmask=dmask, pre_hi=pre_hi):
            u = u_v[pl.ds(v * 16, 16)]
            dig = lax.shift_right_logical(u, jnp.uint32(shift)) & dmask
            if pre_hi is None:
                inc = one
            else:
                uh = lax.shift_right_logical(u, jnp.uint32(hi))
                inc = jnp.where(uh == jnp.broadcast_to(pre_hi, (16,)), one, zero)
            plsc.addupdate_scatter(hist, [plsc.bitcast(dig, jnp.int32)], inc)
            return ()

        lax.fori_loop(0, NVB, acc_body, ())

        pltpu.sync_copy(hist, sh_hist.at[hidx], add=True)
        plsc.subcore_barrier()
        pltpu.sync_copy(sh_hist.at[pl.ds(hbase, 2048)], hist.at[pl.ds(0, 2048)])

        def s_body(j, c, nb=nb):
            above, found, d, g = c
            v = nb - 1 - j
            h = hist[pl.ds(v * 16, 16)]
            s_ge = lax.rev(plsc.cumsum(lax.rev(h, (0,))), (0,))
            tot = above + s_ge
            npos = jnp.max(
                jnp.where(tot >= jnp.broadcast_to(rem, (16,)), one, zero)
                * (iot + 1))
            sel = jnp.where(found == 0, npos, 0)
            lv = jnp.broadcast_to(npos - 1, (16,))
            gv = above + jnp.sum(jnp.where(iot > lv, h, zero))
            dv = v * 16 + npos - 1
            d = jnp.where(sel > 0, dv, d)
            g = jnp.where(sel > 0, gv, g)
            found = jnp.maximum(found, jnp.where(npos > 0, 1, 0))
            above = above + jnp.sum(h)
            return (above, found, d, g)

        _, _, d, g = lax.fori_loop(
            0, nb, s_body,
            (jnp.int32(0), jnp.int32(0), jnp.int32(0), jnp.int32(0)))
        pre = pre | lax.shift_left(
            lax.bitcast_convert_type(d, jnp.uint32), jnp.uint32(shift))
        rem = rem - g

    t_u = pre
    tv = jnp.broadcast_to(t_u, (16,))

    def c_body(v, c):
        ng, ne = c
        u = u_v[pl.ds(v * 16, 16)]
        ng = ng + jnp.sum(jnp.where(u > tv, one, zero))
        ne = ne + jnp.sum(jnp.where(u == tv, one, zero))
        return (ng, ne)

    ng_me, ne_me = lax.fori_loop(0, NVB, c_body,
                                 (jnp.int32(0), jnp.int32(0)))
    cbase = cid * 512
    cnt16[...] = jnp.broadcast_to(ng_me, (16,))
    pltpu.sync_copy(cnt16, sh_cnt.at[pl.ds(cbase + tid * 16, 16)])
    cnt16[...] = jnp.broadcast_to(ne_me, (16,))
    pltpu.sync_copy(cnt16, sh_cnt.at[pl.ds(cbase + 256 + tid * 16, 16)])

    @pl.when(tid == 0)
    def _():
        def pad_body(v, _):
            o = v * 16
            dU[pl.ds(o, 16)] = jnp.full((16,), INT_MIN, jnp.int32)
            dI[pl.ds(o, 16)] = jnp.full((16,), INT_MAX, jnp.int32)
            return ()

        lax.fori_loop(0, K_PAD // 16, pad_body, ())
        pltpu.sync_copy(dU, sh_candU.at[pl.ds(cid * K_PAD, K_PAD)])
        pltpu.sync_copy(dI, sh_candI.at[pl.ds(cid * K_PAD, K_PAD)])

    plsc.subcore_barrier()

    pltpu.sync_copy(sh_cnt.at[pl.ds(cbase, 512)], cntb)
    g_cnt = plsc.load_gather(cntb, [iot * 17])
    e_cnt = plsc.load_gather(cntb, [256 + iot * 17])
    m_tot = jnp.sum(g_cnt)
    ex_g = plsc.cumsum(g_cnt) - g_cnt
    ex_e = plsc.cumsum(e_cnt) - e_cnt
    tsel = jnp.where(iot == jnp.broadcast_to(tid, (16,)), one, zero)
    base_gt = jnp.sum(tsel * ex_g)
    base_eq = jnp.sum(tsel * ex_e)
    mvec = jnp.broadcast_to(m_tot, (16,))

    junk_c = jnp.broadcast_to(cid * K_PAD + K_BG + tid, (16,))
    junk_o = jnp.broadcast_to(cid * OUT_PAD + K_BG + tid, (16,))

    def pf_body(v, _):
        o = v * 16
        candD[pl.ds(o, 16)] = junk_c
        tieD[pl.ds(o, 16)] = junk_o
        return ()

    lax.fori_loop(0, K_PAD // 16, pf_body, ())

    def comp_body(v, carry):
        off, toff, eqc = carry
        o = v * 16
        u = u_v[pl.ds(o, 16)]
        any_rel = jnp.sum(jnp.where(u >= tv, one, zero))

        def do(carry):
            off, toff, eqc = carry
            kv = key_v[pl.ds(o, 16)]
            cv = cls_v[pl.ds(o, 16)]
            gidx = base + o + iot
            m_gt = u > tv
            s = plsc.bitcast(u, jnp.int32) ^ sgn
            dcand = cid * K_PAD + base_gt + off + plsc.cumsum(
                jnp.where(m_gt, one, zero)) - 1
            plsc.store_compressed(candU.at[pl.ds(off, 16)], s, mask=m_gt)
            plsc.store_compressed(candI.at[pl.ds(off, 16)], gidx, mask=m_gt)
            plsc.store_compressed(candC.at[pl.ds(off, 16)], cv, mask=m_gt)
            plsc.store_compressed(candD.at[pl.ds(off, 16)], dcand, mask=m_gt)
            n_gt = jnp.sum(jnp.where(m_gt, one, zero))
            m_eq = u == tv
            eq1 = jnp.where(m_eq, one, zero)
            pos = eqc + plsc.cumsum(eq1) - 1
            dest = mvec + jnp.broadcast_to(base_eq, (16,)) + pos
            keep = jnp.where(m_eq, dest, kvec) < kvec
            plsc.store_compressed(tieV.at[pl.ds(toff, 16)], kv, mask=keep)
            plsc.store_compressed(tieI.at[pl.ds(toff, 16)], gidx, mask=keep)
            plsc.store_compressed(tieC.at[pl.ds(toff, 16)], cv, mask=keep)
            plsc.store_compressed(tieD.at[pl.ds(toff, 16)],
                                  cid * OUT_PAD + dest, mask=keep)
            return (off + n_gt, toff + jnp.sum(jnp.where(keep, one, zero)),
                    eqc + jnp.sum(eq1))

        return lax.cond(any_rel > 0, do, lambda c: c, (off, toff, eqc))

    lax.fori_loop(0, NVB, comp_body,
                  (jnp.int32(0), jnp.int32(0), jnp.int32(0)))

    pltpu.sync_copy(candU, sh_candU.at[candD])
    pltpu.sync_copy(candI, sh_candI.at[candD])
    pltpu.sync_copy(candC, sh_candC.at[candD])
    pltpu.sync_copy(tieV, sh_outV.at[tieD])
    pltpu.sync_copy(tieI, sh_outI.at[tieD])
    pltpu.sync_copy(tieC, sh_outC.at[tieD])
    plsc.subcore_barrier()

    pltpu.sync_copy(sh_candU.at[pl.ds(cid * K_PAD, K_PAD)], dU)
    pltpu.sync_copy(sh_candI.at[pl.ds(cid * K_PAD, K_PAD)], dI)
    pltpu.sync_copy(sh_candC.at[pl.ds(cid * K_PAD, K_PAD)], dC)
    dU[pl.ds(K_BG, 16)] = jnp.full((16,), INT_MIN, jnp.int32)
    dI[pl.ds(K_BG, 16)] = jnp.full((16,), INT_MAX, jnp.int32)

    def rank_block(bb):
        ao = bb * 16
        aU = dU[pl.ds(ao, 16)]
        aI = dI[pl.ds(ao, 16)]
        aC = dC[pl.ds(ao, 16)]

        def b_body(j, acc):
            def r_body(r, acc):
                idx = j * 16 + ((iot + r) & 15)
                bU = plsc.load_gather(dU, [idx])
                bI = plsc.load_gather(dI, [idx])
                tie = jnp.where(bI < aI, one, zero)
                better = jnp.where(bU > aU, one,
                                   jnp.where(bU == aU, tie, zero))
                return acc + better

            return lax.fori_loop(0, 16, r_body, acc)

        rank = lax.fori_loop(0, K_PAD // 16, b_body, zero)
        lanepos = ao + iot
        wmask = jnp.where(lanepos < mvec, rank, kvec) < kvec
        rc = jnp.maximum(jnp.minimum(rank, kvec - 1), 0)
        ui = aU ^ sgn
        vbits = jnp.where(aU >= 0, aU, ~ui)
        st16a[...] = plsc.bitcast(vbits, jnp.float32)
        st16b[...] = aI
        st16c[...] = aC
        st16d[...] = jnp.where(wmask, cid * OUT_PAD + rc, junk_o)
        pltpu.sync_copy(st16a, sh_outV.at[st16d])
        pltpu.sync_copy(st16b, sh_outI.at[st16d])
        pltpu.sync_copy(st16c, sh_outC.at[st16d])

    rank_block(tid)

    @pl.when(tid < (K_PAD // 16) - 16)
    def _():
        rank_block(tid + 16)

    plsc.subcore_barrier()

    @pl.when((cid == 0) & (tid == 0))
    def _():
        pltpu.sync_copy(sh_outV.at[pl.ds(0, K_FG)], tieV.at[pl.ds(0, K_FG)])
        pltpu.sync_copy(sh_outI.at[pl.ds(0, K_FG)], tieI.at[pl.ds(0, K_FG)])
        pltpu.sync_copy(sh_outC.at[pl.ds(0, K_FG)], tieC.at[pl.ds(0, K_FG)])
        pltpu.sync_copy(tieV.at[pl.ds(0, K_FG)], sv_out.at[pl.ds(0, K_FG)])
        pltpu.sync_copy(tieI.at[pl.ds(0, K_FG)], si_out.at[pl.ds(0, K_FG)])
        pltpu.sync_copy(tieC.at[pl.ds(0, K_FG)], scl_out.at[pl.ds(0, K_FG)])

    @pl.when((cid == 1) & (tid == 0))
    def _():
        pltpu.sync_copy(sh_outV.at[pl.ds(OUT_PAD, K_BG)],
                        tieV.at[pl.ds(0, K_BG)])
        pltpu.sync_copy(sh_outI.at[pl.ds(OUT_PAD, K_BG)],
                        tieI.at[pl.ds(0, K_BG)])
        pltpu.sync_copy(sh_outC.at[pl.ds(OUT_PAD, K_BG)],
                        tieC.at[pl.ds(0, K_BG)])
        pltpu.sync_copy(tieV.at[pl.ds(0, K_BG)], sv_out.at[pl.ds(K_FG, K_BG)])
        pltpu.sync_copy(tieI.at[pl.ds(0, K_BG)], si_out.at[pl.ds(K_FG, K_BG)])
        pltpu.sync_copy(tieC.at[pl.ds(0, K_BG)], scl_out.at[pl.ds(K_FG, K_BG)])


@jax.jit
def kernel(proposal_boxes, gt_boxes, scores, gt_classes):
    mesh = plsc.VectorSubcoreMesh(core_axis_name="c", subcore_axis_name="s",
                                  num_cores=2, num_subcores=16)
    f32 = jnp.float32
    i32 = jnp.int32

    pb = jnp.pad(proposal_boxes, ((0, N_PAD - N_PROP), (0, 0)))
    px1, py1, px2, py2 = (pb[:, i] for i in range(4))
    sc = jnp.pad(scores, (0, N_PAD - N_PROP))
    g_rep = [jnp.repeat(gt_boxes[:, i], 16) for i in range(4)]
    gcl_rep = jnp.repeat(gt_classes, 16)

    phase_a = pl.kernel(
        _phase_a_body,
        out_type=(
            jax.ShapeDtypeStruct((N_PAD,), f32),
            jax.ShapeDtypeStruct((N_PAD,), f32),
            jax.ShapeDtypeStruct((N_PAD,), f32),
            jax.ShapeDtypeStruct((N_PAD,), i32),
        ),
        mesh=mesh,
        compiler_params=pltpu.CompilerParams(needs_layout_passes=False),
        scratch_types=[
            pltpu.VMEM((CHUNK,), f32), pltpu.VMEM((CHUNK,), f32),
            pltpu.VMEM((CHUNK,), f32), pltpu.VMEM((CHUNK,), f32),
            pltpu.VMEM((CHUNK,), f32),
            pltpu.VMEM((N_GT * 16,), f32), pltpu.VMEM((N_GT * 16,), f32),
            pltpu.VMEM((N_GT * 16,), f32), pltpu.VMEM((N_GT * 16,), f32),
            pltpu.VMEM((N_GT * 16,), i32),
            pltpu.VMEM((CHUNK,), f32), pltpu.VMEM((CHUNK,), f32),
            pltpu.VMEM((CHUNK,), f32), pltpu.VMEM((CHUNK,), i32),
        ],
    )
    iou_p, fg_key, bg_key, cls = phase_a(
        px1, py1, px2, py2, sc, g_rep[0], g_rep[1], g_rep[2], g_rep[3], gcl_rep)

    phase_b = pl.kernel(
        _phase_b_body,
        out_type=(
            jax.ShapeDtypeStruct((512,), f32),
            jax.ShapeDtypeStruct((512,), i32),
            jax.ShapeDtypeStruct((512,), i32),
        ),
        mesh=mesh,
        compiler_params=pltpu.CompilerParams(needs_layout_passes=False),
        scratch_types=[
            pltpu.VMEM((SLICE,), f32),
            pltpu.VMEM((SLICE,), jnp.uint32),
            pltpu.VMEM((SLICE,), i32),
            pltpu.VMEM((2048,), i32),
            pltpu.VMEM((2048,), i32),
            pltpu.VMEM((K_PAD,), i32),
            pltpu.VMEM((K_PAD,), i32),
            pltpu.VMEM((K_PAD,), i32),
            pltpu.VMEM((K_PAD,), i32),
            pltpu.VMEM((K_PAD,), f32),
            pltpu.VMEM((K_PAD,), i32),
            pltpu.VMEM((K_PAD,), i32),
            pltpu.VMEM((K_PAD,), i32),
            pltpu.VMEM((K_PAD,), i32),
            pltpu.VMEM((K_PAD,), i32),
            pltpu.VMEM((K_PAD,), i32),
            pltpu.VMEM((16,), f32),
            pltpu.VMEM((16,), i32),
            pltpu.VMEM((16,), i32),
            pltpu.VMEM((16,), i32),
            pltpu.VMEM((16,), i32),
            pltpu.VMEM((512,), i32),
            pltpu.VMEM_SHARED((2 * 2048,), i32),
            pltpu.VMEM_SHARED((2 * 512,), i32),
            pltpu.VMEM_SHARED((2 * K_PAD,), i32),
            pltpu.VMEM_SHARED((2 * K_PAD,), i32),
            pltpu.VMEM_SHARED((2 * K_PAD,), i32),
            pltpu.VMEM_SHARED((2 * OUT_PAD,), f32),
            pltpu.VMEM_SHARED((2 * OUT_PAD,), i32),
            pltpu.VMEM_SHARED((2 * OUT_PAD,), i32),
        ],
    )
    sv, si, scl = phase_b(fg_key, bg_key, cls)

    return iou_p[:N_PROP], si, scl, sv


# speedup vs baseline: 2.1435x; 1.0161x over previous
import functools

import jax
import jax.numpy as jnp
from jax import lax
from jax.experimental import pallas as pl
from jax.experimental.pallas import tpu as pltpu
from jax.experimental.pallas import tpu_sc as plsc

N_PROP = 20000
N_PAD = 20480
N_GT = 64
NW = 32
CHUNK = N_PAD // NW
NUM_CLASSES = 80
IOU_THRESH = 0.5
K_FG = 128
K_BG = 384
K_PAD = 400
OUT_PAD = 448
NEG_INF = float("-inf")


def _iota16():
    return lax.iota(jnp.int32, 16)


def _phase_a_body(px1, py1, px2, py2, sc, gx1, gy1, gx2, gy2, gcl,
                  iou_out, fg_out, bg_out, cls_out,
                  px1_v, py1_v, px2_v, py2_v, sc_v,
                  g1_v, g2_v, g3_v, g4_v, gc_v,
                  iou_s, fg_s, bg_s, cls_s):
    wid = lax.axis_index("s") * 2 + lax.axis_index("c")
    base = wid * CHUNK

    pltpu.sync_copy(px1.at[pl.ds(base, CHUNK)], px1_v)
    pltpu.sync_copy(py1.at[pl.ds(base, CHUNK)], py1_v)
    pltpu.sync_copy(px2.at[pl.ds(base, CHUNK)], px2_v)
    pltpu.sync_copy(py2.at[pl.ds(base, CHUNK)], py2_v)
    pltpu.sync_copy(sc.at[pl.ds(base, CHUNK)], sc_v)
    pltpu.sync_copy(gx1, g1_v)
    pltpu.sync_copy(gy1, g2_v)
    pltpu.sync_copy(gx2, g3_v)
    pltpu.sync_copy(gy2, g4_v)
    pltpu.sync_copy(gcl, gc_v)

    def chunk_body(v, _):
        o = v * 16
        p_x1 = px1_v[pl.ds(o, 16)]
        p_y1 = py1_v[pl.ds(o, 16)]
        p_x2 = px2_v[pl.ds(o, 16)]
        p_y2 = py2_v[pl.ds(o, 16)]
        s = sc_v[pl.ds(o, 16)]
        ap = (p_x2 - p_x1) * (p_y2 - p_y1)

        def g_body(g, c):
            bi, bc = c
            go = g * 16
            g_x1 = g1_v[pl.ds(go, 16)]
            g_y1 = g2_v[pl.ds(go, 16)]
            g_x2 = g3_v[pl.ds(go, 16)]
            g_y2 = g4_v[pl.ds(go, 16)]
            g_c = gc_v[pl.ds(go, 16)]
            ag = (g_x2 - g_x1) * (g_y2 - g_y1)
            w = jnp.maximum(jnp.minimum(g_x2, p_x2) - jnp.maximum(g_x1, p_x1), 0.0)
            h = jnp.maximum(jnp.minimum(g_y2, p_y2) - jnp.maximum(g_y1, p_y1), 0.0)
            inter = w * h
            union = ag + ap - inter
            iou = inter / jnp.maximum(union, 1e-9)
            upd = iou > bi
            return (jnp.where(upd, iou, bi), jnp.where(upd, g_c, bc))

        bi0 = jnp.full((16,), -1.0, jnp.float32)
        bc0 = jnp.zeros((16,), jnp.int32)
        bi, bc = lax.fori_loop(0, N_GT, g_body, (bi0, bc0))

        gidx = base + o + _iota16()
        valid = gidx < N_PROP
        matched = bi >= IOU_THRESH
        neg = jnp.full((16,), NEG_INF, jnp.float32)
        fg = jnp.where(
            valid,
            jnp.where(matched, bi, jnp.full((16,), -1.0, jnp.float32)), neg)
        bg = jnp.where(
            valid,
            jnp.where(matched, jnp.full((16,), -1e9, jnp.float32), s), neg)
        cl = jnp.where(matched, bc, jnp.full((16,), NUM_CLASSES, jnp.int32))

        iou_s[pl.ds(o, 16)] = bi
        fg_s[pl.ds(o, 16)] = fg
        bg_s[pl.ds(o, 16)] = bg
        cls_s[pl.ds(o, 16)] = cl
        return ()

    lax.fori_loop(0, CHUNK // 16, chunk_body, ())

    pltpu.sync_copy(iou_s, iou_out.at[pl.ds(base, CHUNK)])
    pltpu.sync_copy(fg_s, fg_out.at[pl.ds(base, CHUNK)])
    pltpu.sync_copy(bg_s, bg_out.at[pl.ds(base, CHUNK)])
    pltpu.sync_copy(cls_s, cls_out.at[pl.ds(base, CHUNK)])


SLICE = N_PAD // 16
NVB = SLICE // 16
INT_MIN = -2147483648
INT_MAX = 2147483647


def _phase_b_body(fg_hbm, bg_hbm, cls_hbm, sv_out, si_out, scl_out,
                  key_v, u_v, cls_v, hist, hidx,
                  candU, candI, candC, candD,
                  tieV, tieI, tieC, tieD,
                  dU, dI, dC, st16a, st16b, st16c, st16d,
                  st16e, st16f, st16g, st16h, cnt32, cntb,
                  sh_hist, sh_cnt, sh_candU, sh_candI, sh_candC,
                  sh_outV, sh_outI, sh_outC, sem):
    cid = lax.axis_index("c")
    tid = lax.axis_index("s")
    k = jnp.where(cid == 0, K_FG, K_BG)
    kvec = jnp.broadcast_to(k, (16,))
    one = jnp.ones((16,), jnp.int32)
    zero = jnp.zeros((16,), jnp.int32)
    iot = _iota16()
    sgn = jnp.int32(INT_MIN)
    base = tid * SLICE

    @pl.when(cid == 0)
    def _():
        h1 = pltpu.async_copy(fg_hbm.at[pl.ds(base, SLICE)], key_v, sem)
        h2 = pltpu.async_copy(cls_hbm.at[pl.ds(base, SLICE)], cls_v, sem)
        h1.wait()
        h2.wait()

    @pl.when(cid == 1)
    def _():
        h1 = pltpu.async_copy(bg_hbm.at[pl.ds(base, SLICE)], key_v, sem)
        h2 = pltpu.async_copy(cls_hbm.at[pl.ds(base, SLICE)], cls_v, sem)
        h1.wait()
        h2.wait()

    def z0_body(v, _):
        hist[pl.ds(v * 16, 16)] = zero
        return ()

    lax.fori_loop(0, 128, z0_body, ())
    hbase = cid * 6144
    hz = pltpu.async_copy(hist.at[pl.ds(0, 384)],
                          sh_hist.at[pl.ds(hbase + tid * 384, 384)], sem)

    def u_body(v, _):
        o = v * 16
        b = plsc.bitcast(key_v[pl.ds(o, 16)], jnp.int32)
        u = jnp.where(b < 0, ~b, b ^ sgn)
        u_v[pl.ds(o, 16)] = plsc.bitcast(u, jnp.uint32)
        return ()

    lax.fori_loop(0, NVB, u_body, ())
    hz.wait()
    plsc.subcore_barrier()

    pre = jnp.uint32(0)
    rem = k
    for pnum, (shift, width) in enumerate(((21, 11), (10, 11), (0, 10))):
        nbins = 1 << width
        nb = nbins // 16
        pbase = hbase + pnum * 2048

        def z_body(v, _, pbase=pbase):
            hidx[pl.ds(v * 16, 16)] = pbase + v * 16 + iot
            if pnum:
                hist[pl.ds(v * 16, 16)] = zero
            return ()

        lax.fori_loop(0, 128, z_body, ())

        dmask = jnp.uint32(nbins - 1)
        hi = shift + width
        pre_hi = lax.shift_right_logical(pre, jnp.uint32(hi)) if hi < 32 else None

        def acc_body(v, _, shift=shift, hi=hi, dmask=dmask, pre_hi=pre_hi):
            u = u_v[pl.ds(v * 16, 16)]
            dig = lax.shift_right_logical(u, jnp.uint32(shift)) & dmask
            if pre_hi is None:
                inc = one
            else:
                uh = lax.shift_right_logical(u, jnp.uint32(hi))
                inc = jnp.where(uh == jnp.broadcast_to(pre_hi, (16,)), one, zero)
            plsc.addupdate_scatter(hist, [plsc.bitcast(dig, jnp.int32)], inc)
            return ()

        lax.fori_loop(0, NVB, acc_body, ())

        pltpu.sync_copy(hist, sh_hist.at[hidx], add=True)
        plsc.subcore_barrier()
        pltpu.sync_copy(sh_hist.at[pl.ds(pbase, 2048)], hist.at[pl.ds(0, 2048)])

        def s_body(j, c, nb=nb):
            above, found, d, g = c
            v = nb - 1 - j
            h = hist[pl.ds(v * 16, 16)]
            s_ge = lax.rev(plsc.cumsum(lax.rev(h, (0,))), (0,))
            tot = above + s_ge
            npos = jnp.max(
                jnp.where(tot >= jnp.broadcast_to(rem, (16,)), one, zero)
                * (iot + 1))
            sel = jnp.where(found == 0, npos, 0)
            lv = jnp.broadcast_to(npos - 1, (16,))
            gv = above + jnp.sum(jnp.where(iot > lv, h, zero))
            dv = v * 16 + npos - 1
            d = jnp.where(sel > 0, dv, d)
            g = jnp.where(sel > 0, gv, g)
            found = jnp.maximum(found, jnp.where(npos > 0, 1, 0))
            above = above + jnp.sum(h)
            return (above, found, d, g)

        _, _, d, g = lax.fori_loop(
            0, nb, s_body,
            (jnp.int32(0), jnp.int32(0), jnp.int32(0), jnp.int32(0)))
        pre = pre | lax.shift_left(
            lax.bitcast_convert_type(d, jnp.uint32), jnp.uint32(shift))
        rem = rem - g

    t_u = pre
    tv = jnp.broadcast_to(t_u, (16,))

    def c_body(v, c):
        ng, ne = c
        u = u_v[pl.ds(v * 16, 16)]
        ng = ng + jnp.sum(jnp.where(u > tv, one, zero))
        ne = ne + jnp.sum(jnp.where(u == tv, one, zero))
        return (ng, ne)

    ng_me, ne_me = lax.fori_loop(0, NVB, c_body,
                                 (jnp.int32(0), jnp.int32(0)))
    cbase = cid * 512
    cnt32[pl.ds(0, 16)] = jnp.broadcast_to(ng_me, (16,))
    cnt32[pl.ds(16, 16)] = jnp.broadcast_to(ne_me, (16,))
    hc = pltpu.async_copy(cnt32, sh_cnt.at[pl.ds(cbase + tid * 32, 32)], sem)

    @pl.when(tid == 0)
    def _():
        def pad_body(v, _):
            o = v * 16
            dU[pl.ds(o, 16)] = jnp.full((16,), INT_MIN, jnp.int32)
            dI[pl.ds(o, 16)] = jnp.full((16,), INT_MAX, jnp.int32)
            return ()

        lax.fori_loop(0, K_PAD // 16, pad_body, ())
        p1 = pltpu.async_copy(dU, sh_candU.at[pl.ds(cid * K_PAD, K_PAD)], sem)
        p2 = pltpu.async_copy(dI, sh_candI.at[pl.ds(cid * K_PAD, K_PAD)], sem)
        p1.wait()
        p2.wait()

    hc.wait()
    plsc.subcore_barrier()

    pltpu.sync_copy(sh_cnt.at[pl.ds(cbase, 512)], cntb)
    g_cnt = plsc.load_gather(cntb, [iot * 33])
    e_cnt = plsc.load_gather(cntb, [iot * 33 + 16])
    m_tot = jnp.sum(g_cnt)
    ex_g = plsc.cumsum(g_cnt) - g_cnt
    ex_e = plsc.cumsum(e_cnt) - e_cnt
    tsel = jnp.where(iot == jnp.broadcast_to(tid, (16,)), one, zero)
    base_gt = jnp.sum(tsel * ex_g)
    base_eq = jnp.sum(tsel * ex_e)
    mvec = jnp.broadcast_to(m_tot, (16,))

    junk_c = jnp.broadcast_to(cid * K_PAD + K_BG + tid, (16,))
    junk_o = jnp.broadcast_to(cid * OUT_PAD + K_BG + tid, (16,))

    def pf_body(v, _):
        o = v * 16
        candD[pl.ds(o, 16)] = junk_c
        tieD[pl.ds(o, 16)] = junk_o
        return ()

    lax.fori_loop(0, K_PAD // 16, pf_body, ())

    def comp_body(v, carry):
        off, toff, eqc = carry
        o = v * 16
        u = u_v[pl.ds(o, 16)]
        any_rel = jnp.sum(jnp.where(u >= tv, one, zero))

        def do(carry):
            off, toff, eqc = carry
            kv = key_v[pl.ds(o, 16)]
            cv = cls_v[pl.ds(o, 16)]
            gidx = base + o + iot
            m_gt = u > tv
            s = plsc.bitcast(u, jnp.int32) ^ sgn
            dcand = cid * K_PAD + base_gt + off + plsc.cumsum(
                jnp.where(m_gt, one, zero)) - 1
            plsc.store_compressed(candU.at[pl.ds(off, 16)], s, mask=m_gt)
            plsc.store_compressed(candI.at[pl.ds(off, 16)], gidx, mask=m_gt)
            plsc.store_compressed(candC.at[pl.ds(off, 16)], cv, mask=m_gt)
            plsc.store_compressed(candD.at[pl.ds(off, 16)], dcand, mask=m_gt)
            n_gt = jnp.sum(jnp.where(m_gt, one, zero))
            m_eq = u == tv
            eq1 = jnp.where(m_eq, one, zero)
            pos = eqc + plsc.cumsum(eq1) - 1
            dest = mvec + jnp.broadcast_to(base_eq, (16,)) + pos
            keep = jnp.where(m_eq, dest, kvec) < kvec
            plsc.store_compressed(tieV.at[pl.ds(toff, 16)], kv, mask=keep)
            plsc.store_compressed(tieI.at[pl.ds(toff, 16)], gidx, mask=keep)
            plsc.store_compressed(tieC.at[pl.ds(toff, 16)], cv, mask=keep)
            plsc.store_compressed(tieD.at[pl.ds(toff, 16)],
                                  cid * OUT_PAD + dest, mask=keep)
            return (off + n_gt, toff + jnp.sum(jnp.where(keep, one, zero)),
                    eqc + jnp.sum(eq1))

        return lax.cond(any_rel > 0, do, lambda c: c, (off, toff, eqc))

    lax.fori_loop(0, NVB, comp_body,
                  (jnp.int32(0), jnp.int32(0), jnp.int32(0)))

    hs = [pltpu.async_copy(candU, sh_candU.at[candD], sem),
          pltpu.async_copy(candI, sh_candI.at[candD], sem),
          pltpu.async_copy(candC, sh_candC.at[candD], sem),
          pltpu.async_copy(tieV, sh_outV.at[tieD], sem),
          pltpu.async_copy(tieI, sh_outI.at[tieD], sem),
          pltpu.async_copy(tieC, sh_outC.at[tieD], sem)]
    for h in hs:
        h.wait()
    plsc.subcore_barrier()

    hs = [pltpu.async_copy(sh_candU.at[pl.ds(cid * K_PAD, K_PAD)], dU, sem),
          pltpu.async_copy(sh_candI.at[pl.ds(cid * K_PAD, K_PAD)], dI, sem),
          pltpu.async_copy(sh_candC.at[pl.ds(cid * K_PAD, K_PAD)], dC, sem)]
    for h in hs:
        h.wait()
    dU[pl.ds(K_BG, 16)] = jnp.full((16,), INT_MIN, jnp.int32)
    dI[pl.ds(K_BG, 16)] = jnp.full((16,), INT_MAX, jnp.int32)

    def rank_block(bb, sta, stb, stc, std):
        ao = bb * 16
        aU = dU[pl.ds(ao, 16)]
        aI = dI[pl.ds(ao, 16)]
        aC = dC[pl.ds(ao, 16)]

        def b_body(j, acc):
            def r_body(r, acc):
                idx = j * 16 + ((iot + r) & 15)
                bU = plsc.load_gather(dU, [idx])
                bI = plsc.load_gather(dI, [idx])
                tie = jnp.where(bI < aI, one, zero)
                better = jnp.where(bU > aU, one,
                                   jnp.where(bU == aU, tie, zero))
                return acc + better

            return lax.fori_loop(0, 16, r_body, acc)

        rank = lax.fori_loop(0, K_PAD // 16, b_body, zero)
        lanepos = ao + iot
        wmask = jnp.where(lanepos < mvec, rank, kvec) < kvec
        rc = jnp.maximum(jnp.minimum(rank, kvec - 1), 0)
        ui = aU ^ sgn
        vbits = jnp.where(aU >= 0, aU, ~ui)
        sta[...] = plsc.bitcast(vbits, jnp.float32)
        stb[...] = aI
        stc[...] = aC
        std[...] = jnp.where(wmask, cid * OUT_PAD + rc, junk_o)
        hs = [pltpu.async_copy(sta, sh_outV.at[std], sem),
              pltpu.async_copy(stb, sh_outI.at[std], sem),
              pltpu.async_copy(stc, sh_outC.at[std], sem)]
        for h in hs:
            h.wait()

    rank_block(tid, st16a, st16b, st16c, st16d)

    @pl.when(tid < (K_PAD // 16) - 16)
    def _():
        rank_block(tid + 16, st16e, st16f, st16g, st16h)

    plsc.subcore_barrier()

    @pl.when((cid == 0) & (tid == 0))
    def _():
        hs = [pltpu.async_copy(sh_outV.at[pl.ds(0, K_FG)],
                               tieV.at[pl.ds(0, K_FG)], sem),
              pltpu.async_copy(sh_outI.at[pl.ds(0, K_FG)],
                               tieI.at[pl.ds(0, K_FG)], sem),
              pltpu.async_copy(sh_outC.at[pl.ds(0, K_FG)],
                               tieC.at[pl.ds(0, K_FG)], sem)]
        for h in hs:
            h.wait()
        hs = [pltpu.async_copy(tieV.at[pl.ds(0, K_FG)],
                               sv_out.at[pl.ds(0, K_FG)], sem),
              pltpu.async_copy(tieI.at[pl.ds(0, K_FG)],
                               si_out.at[pl.ds(0, K_FG)], sem),
              pltpu.async_copy(tieC.at[pl.ds(0, K_FG)],
                               scl_out.at[pl.ds(0, K_FG)], sem)]
        for h in hs:
            h.wait()

    @pl.when((cid == 1) & (tid == 0))
    def _():
        hs = [pltpu.async_copy(sh_outV.at[pl.ds(OUT_PAD, K_BG)],
                               tieV.at[pl.ds(0, K_BG)], sem),
              pltpu.async_copy(sh_outI.at[pl.ds(OUT_PAD, K_BG)],
                               tieI.at[pl.ds(0, K_BG)], sem),
              pltpu.async_copy(sh_outC.at[pl.ds(OUT_PAD, K_BG)],
                               tieC.at[pl.ds(0, K_BG)], sem)]
        for h in hs:
            h.wait()
        hs = [pltpu.async_copy(tieV.at[pl.ds(0, K_BG)],
                               sv_out.at[pl.ds(K_FG, K_BG)], sem),
              pltpu.async_copy(tieI.at[pl.ds(0, K_BG)],
                               si_out.at[pl.ds(K_FG, K_BG)], sem),
              pltpu.async_copy(tieC.at[pl.ds(0, K_BG)],
                               scl_out.at[pl.ds(K_FG, K_BG)], sem)]
        for h in hs:
            h.wait()


@jax.jit
def kernel(proposal_boxes, gt_boxes, scores, gt_classes):
    mesh = plsc.VectorSubcoreMesh(core_axis_name="c", subcore_axis_name="s",
                                  num_cores=2, num_subcores=16)
    f32 = jnp.float32
    i32 = jnp.int32

    pb = jnp.pad(proposal_boxes, ((0, N_PAD - N_PROP), (0, 0)))
    px1, py1, px2, py2 = (pb[:, i] for i in range(4))
    sc = jnp.pad(scores, (0, N_PAD - N_PROP))
    g_rep = [jnp.repeat(gt_boxes[:, i], 16) for i in range(4)]
    gcl_rep = jnp.repeat(gt_classes, 16)

    phase_a = pl.kernel(
        _phase_a_body,
        out_type=(
            jax.ShapeDtypeStruct((N_PAD,), f32),
            jax.ShapeDtypeStruct((N_PAD,), f32),
            jax.ShapeDtypeStruct((N_PAD,), f32),
            jax.ShapeDtypeStruct((N_PAD,), i32),
        ),
        mesh=mesh,
        compiler_params=pltpu.CompilerParams(needs_layout_passes=False),
        scratch_types=[
            pltpu.VMEM((CHUNK,), f32), pltpu.VMEM((CHUNK,), f32),
            pltpu.VMEM((CHUNK,), f32), pltpu.VMEM((CHUNK,), f32),
            pltpu.VMEM((CHUNK,), f32),
            pltpu.VMEM((N_GT * 16,), f32), pltpu.VMEM((N_GT * 16,), f32),
            pltpu.VMEM((N_GT * 16,), f32), pltpu.VMEM((N_GT * 16,), f32),
            pltpu.VMEM((N_GT * 16,), i32),
            pltpu.VMEM((CHUNK,), f32), pltpu.VMEM((CHUNK,), f32),
            pltpu.VMEM((CHUNK,), f32), pltpu.VMEM((CHUNK,), i32),
        ],
    )
    iou_p, fg_key, bg_key, cls = phase_a(
        px1, py1, px2, py2, sc, g_rep[0], g_rep[1], g_rep[2], g_rep[3], gcl_rep)

    phase_b = pl.kernel(
        _phase_b_body,
        out_type=(
            jax.ShapeDtypeStruct((512,), f32),
            jax.ShapeDtypeStruct((512,), i32),
            jax.ShapeDtypeStruct((512,), i32),
        ),
        mesh=mesh,
        compiler_params=pltpu.CompilerParams(needs_layout_passes=False),
        scratch_types=[
            pltpu.VMEM((SLICE,), f32),
            pltpu.VMEM((SLICE,), jnp.uint32),
            pltpu.VMEM((SLICE,), i32),
            pltpu.VMEM((2048,), i32),
            pltpu.VMEM((2048,), i32),
            pltpu.VMEM((K_PAD,), i32),
            pltpu.VMEM((K_PAD,), i32),
            pltpu.VMEM((K_PAD,), i32),
            pltpu.VMEM((K_PAD,), i32),
            pltpu.VMEM((K_PAD,), f32),
            pltpu.VMEM((K_PAD,), i32),
            pltpu.VMEM((K_PAD,), i32),
            pltpu.VMEM((K_PAD,), i32),
            pltpu.VMEM((K_PAD,), i32),
            pltpu.VMEM((K_PAD,), i32),
            pltpu.VMEM((K_PAD,), i32),
            pltpu.VMEM((16,), f32),
            pltpu.VMEM((16,), i32),
            pltpu.VMEM((16,), i32),
            pltpu.VMEM((16,), i32),
            pltpu.VMEM((16,), f32),
            pltpu.VMEM((16,), i32),
            pltpu.VMEM((16,), i32),
            pltpu.VMEM((16,), i32),
            pltpu.VMEM((32,), i32),
            pltpu.VMEM((512,), i32),
            pltpu.VMEM_SHARED((2 * 6144,), i32),
            pltpu.VMEM_SHARED((2 * 512,), i32),
            pltpu.VMEM_SHARED((2 * K_PAD,), i32),
            pltpu.VMEM_SHARED((2 * K_PAD,), i32),
            pltpu.VMEM_SHARED((2 * K_PAD,), i32),
            pltpu.VMEM_SHARED((2 * OUT_PAD,), f32),
            pltpu.VMEM_SHARED((2 * OUT_PAD,), i32),
            pltpu.VMEM_SHARED((2 * OUT_PAD,), i32),
            pltpu.SemaphoreType.DMA,
        ],
    )
    sv, si, scl = phase_b(fg_key, bg_key, cls)

    return iou_p[:N_PROP], si, scl, sv


# speedup vs baseline: 2.3505x; 1.0966x over previous
import functools

import jax
import jax.numpy as jnp
from jax import lax
from jax.experimental import pallas as pl
from jax.experimental.pallas import tpu as pltpu
from jax.experimental.pallas import tpu_sc as plsc

N_PROP = 20000
N_PAD = 20480
N_GT = 64
NW = 32
CHUNK = N_PAD // NW
NUM_CLASSES = 80
IOU_THRESH = 0.5
K_FG = 128
K_BG = 384
K_PAD = 400
OUT_PAD = 448
NEG_INF = float("-inf")


def _iota16():
    return lax.iota(jnp.int32, 16)


def _phase_a_body(px1, py1, px2, py2, sc, gx1, gy1, gx2, gy2, gcl,
                  iou_out, fg_out, bg_out, cls_out,
                  px1_v, py1_v, px2_v, py2_v, sc_v,
                  g1_v, g2_v, g3_v, g4_v, gc_v,
                  iou_s, fg_s, bg_s, cls_s, ag_v):
    wid = lax.axis_index("s") * 2 + lax.axis_index("c")
    base = wid * CHUNK

    pltpu.sync_copy(px1.at[pl.ds(base, CHUNK)], px1_v)
    pltpu.sync_copy(py1.at[pl.ds(base, CHUNK)], py1_v)
    pltpu.sync_copy(px2.at[pl.ds(base, CHUNK)], px2_v)
    pltpu.sync_copy(py2.at[pl.ds(base, CHUNK)], py2_v)
    pltpu.sync_copy(sc.at[pl.ds(base, CHUNK)], sc_v)
    pltpu.sync_copy(gx1, g1_v)
    pltpu.sync_copy(gy1, g2_v)
    pltpu.sync_copy(gx2, g3_v)
    pltpu.sync_copy(gy2, g4_v)
    pltpu.sync_copy(gcl, gc_v)

    def area_body(g, _):
        go = g * 16
        ag_v[pl.ds(go, 16)] = ((g3_v[pl.ds(go, 16)] - g1_v[pl.ds(go, 16)])
                               * (g4_v[pl.ds(go, 16)] - g2_v[pl.ds(go, 16)]))
        return ()

    lax.fori_loop(0, N_GT, area_body, ())

    def chunk_body(v, _):
        o = v * 16
        p_x1 = px1_v[pl.ds(o, 16)]
        p_y1 = py1_v[pl.ds(o, 16)]
        p_x2 = px2_v[pl.ds(o, 16)]
        p_y2 = py2_v[pl.ds(o, 16)]
        s = sc_v[pl.ds(o, 16)]
        ap = (p_x2 - p_x1) * (p_y2 - p_y1)

        def g_body(g, c):
            bi, bc = c
            go = g * 16
            g_x1 = g1_v[pl.ds(go, 16)]
            g_y1 = g2_v[pl.ds(go, 16)]
            g_x2 = g3_v[pl.ds(go, 16)]
            g_y2 = g4_v[pl.ds(go, 16)]
            g_c = gc_v[pl.ds(go, 16)]
            ag = ag_v[pl.ds(go, 16)]
            w = jnp.maximum(jnp.minimum(g_x2, p_x2) - jnp.maximum(g_x1, p_x1), 0.0)
            h = jnp.maximum(jnp.minimum(g_y2, p_y2) - jnp.maximum(g_y1, p_y1), 0.0)
            inter = w * h
            union = ag + ap - inter
            iou = inter / jnp.maximum(union, 1e-9)
            upd = iou > bi
            return (jnp.where(upd, iou, bi), jnp.where(upd, g_c, bc))

        bi0 = jnp.full((16,), -1.0, jnp.float32)
        bc0 = jnp.zeros((16,), jnp.int32)
        bi, bc = lax.fori_loop(0, N_GT, g_body, (bi0, bc0))

        gidx = base + o + _iota16()
        valid = gidx < N_PROP
        matched = bi >= IOU_THRESH
        neg = jnp.full((16,), NEG_INF, jnp.float32)
        fg = jnp.where(
            valid,
            jnp.where(matched, bi, jnp.full((16,), -1.0, jnp.float32)), neg)
        bg = jnp.where(
            valid,
            jnp.where(matched, jnp.full((16,), -1e9, jnp.float32), s), neg)
        cl = jnp.where(matched, bc, jnp.full((16,), NUM_CLASSES, jnp.int32))

        iou_s[pl.ds(o, 16)] = bi
        fg_s[pl.ds(o, 16)] = fg
        bg_s[pl.ds(o, 16)] = bg
        cls_s[pl.ds(o, 16)] = cl
        return ()

    lax.fori_loop(0, CHUNK // 16, chunk_body, ())

    pltpu.sync_copy(iou_s, iou_out.at[pl.ds(base, CHUNK)])
    pltpu.sync_copy(fg_s, fg_out.at[pl.ds(base, CHUNK)])
    pltpu.sync_copy(bg_s, bg_out.at[pl.ds(base, CHUNK)])
    pltpu.sync_copy(cls_s, cls_out.at[pl.ds(base, CHUNK)])


SLICE = N_PAD // 16
NVB = SLICE // 16
INT_MIN = -2147483648
INT_MAX = 2147483647


def _phase_b_body(fg_hbm, bg_hbm, cls_hbm, sv_out, si_out, scl_out,
                  key_v, u_v, cls_v, hist, hidx,
                  candU, candI, candC, candD,
                  tieV, tieI, tieC, tieD,
                  dU, dI, dC, st16a, st16b, st16c, st16d,
                  st16e, st16f, st16g, st16h, cnt32, cntb,
                  sh_hist, sh_cnt, sh_candU, sh_candI, sh_candC,
                  sh_outV, sh_outI, sh_outC, sem):
    cid = lax.axis_index("c")
    tid = lax.axis_index("s")
    k = jnp.where(cid == 0, K_FG, K_BG)
    kvec = jnp.broadcast_to(k, (16,))
    one = jnp.ones((16,), jnp.int32)
    zero = jnp.zeros((16,), jnp.int32)
    iot = _iota16()
    sgn = jnp.int32(INT_MIN)
    base = tid * SLICE

    @pl.when(cid == 0)
    def _():
        h1 = pltpu.async_copy(fg_hbm.at[pl.ds(base, SLICE)], key_v, sem)
        h2 = pltpu.async_copy(cls_hbm.at[pl.ds(base, SLICE)], cls_v, sem)
        h1.wait()
        h2.wait()

    @pl.when(cid == 1)
    def _():
        h1 = pltpu.async_copy(bg_hbm.at[pl.ds(base, SLICE)], key_v, sem)
        h2 = pltpu.async_copy(cls_hbm.at[pl.ds(base, SLICE)], cls_v, sem)
        h1.wait()
        h2.wait()

    def z0_body(v, _):
        hist[pl.ds(v * 16, 16)] = zero
        return ()

    lax.fori_loop(0, 128, z0_body, ())
    hbase = cid * 6144
    hz = pltpu.async_copy(hist.at[pl.ds(0, 384)],
                          sh_hist.at[pl.ds(hbase + tid * 384, 384)], sem)

    def u_body(v, _):
        o = v * 16
        b = plsc.bitcast(key_v[pl.ds(o, 16)], jnp.int32)
        u = jnp.where(b < 0, ~b, b ^ sgn)
        u_v[pl.ds(o, 16)] = plsc.bitcast(u, jnp.uint32)
        return ()

    lax.fori_loop(0, NVB, u_body, ())
    hz.wait()
    plsc.subcore_barrier()

    pre = jnp.uint32(0)
    rem = k
    for pnum, (shift, width) in enumerate(((21, 11), (10, 11), (0, 10))):
        nbins = 1 << width
        nb = nbins // 16
        pbase = hbase + pnum * 2048

        def z_body(v, _, pbase=pbase):
            hidx[pl.ds(v * 16, 16)] = pbase + v * 16 + iot
            if pnum:
                hist[pl.ds(v * 16, 16)] = zero
            return ()

        lax.fori_loop(0, 128, z_body, ())

        dmask = jnp.uint32(nbins - 1)
        hi = shift + width
        pre_hi = lax.shift_right_logical(pre, jnp.uint32(hi)) if hi < 32 else None

        def acc_body(v, _, shift=shift, hi=hi, dmask=dmask, pre_hi=pre_hi):
            u = u_v[pl.ds(v * 16, 16)]
            dig = lax.shift_right_logical(u, jnp.uint32(shift)) & dmask
            if pre_hi is None:
                inc = one
            else:
                uh = lax.shift_right_logical(u, jnp.uint32(hi))
                inc = jnp.where(uh == jnp.broadcast_to(pre_hi, (16,)), one, zero)
            plsc.addupdate_scatter(hist, [plsc.bitcast(dig, jnp.int32)], inc)
            return ()

        lax.fori_loop(0, NVB, acc_body, ())

        pltpu.sync_copy(hist, sh_hist.at[hidx], add=True)
        plsc.subcore_barrier()
        pltpu.sync_copy(sh_hist.at[pl.ds(pbase, 2048)], hist.at[pl.ds(0, 2048)])

        def s_cond(c, nb=nb):
            j, above, found, d, g = c
            return (found == 0) & (j < nb)

        def s_step(c, nb=nb):
            j, above, found, d, g = c
            v = nb - 1 - j
            h = hist[pl.ds(v * 16, 16)]
            s_ge = lax.rev(plsc.cumsum(lax.rev(h, (0,))), (0,))
            tot = above + s_ge
            npos = jnp.max(
                jnp.where(tot >= jnp.broadcast_to(rem, (16,)), one, zero)
                * (iot + 1))
            lv = jnp.broadcast_to(npos - 1, (16,))
            gv = above + jnp.sum(jnp.where(iot > lv, h, zero))
            dv = v * 16 + npos - 1
            d = jnp.where(npos > 0, dv, d)
            g = jnp.where(npos > 0, gv, g)
            found = jnp.where(npos > 0, 1, 0)
            above = above + jnp.sum(h)
            return (j + 1, above, found, d, g)

        _, _, _, d, g = lax.while_loop(
            s_cond, s_step,
            (jnp.int32(0), jnp.int32(0), jnp.int32(0), jnp.int32(0),
             jnp.int32(0)))
        pre = pre | lax.shift_left(
            lax.bitcast_convert_type(d, jnp.uint32), jnp.uint32(shift))
        rem = rem - g

    t_u = pre
    tv = jnp.broadcast_to(t_u, (16,))

    def c_body(v, c):
        ng, ne = c
        u = u_v[pl.ds(v * 16, 16)]
        ng = ng + jnp.sum(jnp.where(u > tv, one, zero))
        ne = ne + jnp.sum(jnp.where(u == tv, one, zero))
        return (ng, ne)

    ng_me, ne_me = lax.fori_loop(0, NVB, c_body,
                                 (jnp.int32(0), jnp.int32(0)))
    cbase = cid * 512
    cnt32[pl.ds(0, 16)] = jnp.broadcast_to(ng_me, (16,))
    cnt32[pl.ds(16, 16)] = jnp.broadcast_to(ne_me, (16,))
    hc = pltpu.async_copy(cnt32, sh_cnt.at[pl.ds(cbase + tid * 32, 32)], sem)

    @pl.when(tid == 0)
    def _():
        def pad_body(v, _):
            o = v * 16
            dU[pl.ds(o, 16)] = jnp.full((16,), INT_MIN, jnp.int32)
            dI[pl.ds(o, 16)] = jnp.full((16,), INT_MAX, jnp.int32)
            return ()

        lax.fori_loop(0, K_PAD // 16, pad_body, ())
        p1 = pltpu.async_copy(dU, sh_candU.at[pl.ds(cid * K_PAD, K_PAD)], sem)
        p2 = pltpu.async_copy(dI, sh_candI.at[pl.ds(cid * K_PAD, K_PAD)], sem)
        p1.wait()
        p2.wait()

    hc.wait()
    plsc.subcore_barrier()

    pltpu.sync_copy(sh_cnt.at[pl.ds(cbase, 512)], cntb)
    g_cnt = plsc.load_gather(cntb, [iot * 33])
    e_cnt = plsc.load_gather(cntb, [iot * 33 + 16])
    m_tot = jnp.sum(g_cnt)
    ex_g = plsc.cumsum(g_cnt) - g_cnt
    ex_e = plsc.cumsum(e_cnt) - e_cnt
    tsel = jnp.where(iot == jnp.broadcast_to(tid, (16,)), one, zero)
    base_gt = jnp.sum(tsel * ex_g)
    base_eq = jnp.sum(tsel * ex_e)
    mvec = jnp.broadcast_to(m_tot, (16,))

    junk_c = jnp.broadcast_to(cid * K_PAD + K_BG + tid, (16,))
    junk_o = jnp.broadcast_to(cid * OUT_PAD + K_BG + tid, (16,))

    def pf_body(v, _):
        o = v * 16
        candD[pl.ds(o, 16)] = junk_c
        tieD[pl.ds(o, 16)] = junk_o
        return ()

    lax.fori_loop(0, K_PAD // 16, pf_body, ())

    def comp_body(v, carry):
        off, toff, eqc = carry
        o = v * 16
        u = u_v[pl.ds(o, 16)]
        any_rel = jnp.sum(jnp.where(u >= tv, one, zero))

        def do(carry):
            off, toff, eqc = carry
            kv = key_v[pl.ds(o, 16)]
            cv = cls_v[pl.ds(o, 16)]
            gidx = base + o + iot
            m_gt = u > tv
            s = plsc.bitcast(u, jnp.int32) ^ sgn
            dcand = cid * K_PAD + base_gt + off + plsc.cumsum(
                jnp.where(m_gt, one, zero)) - 1
            plsc.store_compressed(candU.at[pl.ds(off, 16)], s, mask=m_gt)
            plsc.store_compressed(candI.at[pl.ds(off, 16)], gidx, mask=m_gt)
            plsc.store_compressed(candC.at[pl.ds(off, 16)], cv, mask=m_gt)
            plsc.store_compressed(candD.at[pl.ds(off, 16)], dcand, mask=m_gt)
            n_gt = jnp.sum(jnp.where(m_gt, one, zero))
            m_eq = u == tv
            eq1 = jnp.where(m_eq, one, zero)
            pos = eqc + plsc.cumsum(eq1) - 1
            dest = mvec + jnp.broadcast_to(base_eq, (16,)) + pos
            keep = jnp.where(m_eq, dest, kvec) < kvec
            plsc.store_compressed(tieV.at[pl.ds(toff, 16)], kv, mask=keep)
            plsc.store_compressed(tieI.at[pl.ds(toff, 16)], gidx, mask=keep)
            plsc.store_compressed(tieC.at[pl.ds(toff, 16)], cv, mask=keep)
            plsc.store_compressed(tieD.at[pl.ds(toff, 16)],
                                  cid * OUT_PAD + dest, mask=keep)
            return (off + n_gt, toff + jnp.sum(jnp.where(keep, one, zero)),
                    eqc + jnp.sum(eq1))

        return lax.cond(any_rel > 0, do, lambda c: c, (off, toff, eqc))

    lax.fori_loop(0, NVB, comp_body,
                  (jnp.int32(0), jnp.int32(0), jnp.int32(0)))

    hs = [pltpu.async_copy(candU, sh_candU.at[candD], sem),
          pltpu.async_copy(candI, sh_candI.at[candD], sem),
          pltpu.async_copy(candC, sh_candC.at[candD], sem),
          pltpu.async_copy(tieV, sh_outV.at[tieD], sem),
          pltpu.async_copy(tieI, sh_outI.at[tieD], sem),
          pltpu.async_copy(tieC, sh_outC.at[tieD], sem)]
    for h in hs:
        h.wait()
    plsc.subcore_barrier()

    hs = [pltpu.async_copy(sh_candU.at[pl.ds(cid * K_PAD, K_PAD)], dU, sem),
          pltpu.async_copy(sh_candI.at[pl.ds(cid * K_PAD, K_PAD)], dI, sem),
          pltpu.async_copy(sh_candC.at[pl.ds(cid * K_PAD, K_PAD)], dC, sem)]
    for h in hs:
        h.wait()
    dU[pl.ds(K_BG, 16)] = jnp.full((16,), INT_MIN, jnp.int32)
    dI[pl.ds(K_BG, 16)] = jnp.full((16,), INT_MAX, jnp.int32)

    def rank_block(bb, sta, stb, stc, std):
        ao = bb * 16
        aU = dU[pl.ds(ao, 16)]
        aI = dI[pl.ds(ao, 16)]
        aC = dC[pl.ds(ao, 16)]

        def b_body(j, acc):
            def r_body(r, acc):
                idx = j * 16 + ((iot + r) & 15)
                bU = plsc.load_gather(dU, [idx])
                bI = plsc.load_gather(dI, [idx])
                tie = jnp.where(bI < aI, one, zero)
                better = jnp.where(bU > aU, one,
                                   jnp.where(bU == aU, tie, zero))
                return acc + better

            return lax.fori_loop(0, 16, r_body, acc)

        rank = lax.fori_loop(0, K_PAD // 16, b_body, zero)
        lanepos = ao + iot
        wmask = jnp.where(lanepos < mvec, rank, kvec) < kvec
        rc = jnp.maximum(jnp.minimum(rank, kvec - 1), 0)
        ui = aU ^ sgn
        vbits = jnp.where(aU >= 0, aU, ~ui)
        sta[...] = plsc.bitcast(vbits, jnp.float32)
        stb[...] = aI
        stc[...] = aC
        std[...] = jnp.where(wmask, cid * OUT_PAD + rc, junk_o)
        hs = [pltpu.async_copy(sta, sh_outV.at[std], sem),
              pltpu.async_copy(stb, sh_outI.at[std], sem),
              pltpu.async_copy(stc, sh_outC.at[std], sem)]
        for h in hs:
            h.wait()

    rank_block(tid, st16a, st16b, st16c, st16d)

    @pl.when(tid < (K_PAD // 16) - 16)
    def _():
        rank_block(tid + 16, st16e, st16f, st16g, st16h)

    plsc.subcore_barrier()

    @pl.when((cid == 0) & (tid == 0))
    def _():
        hs = [pltpu.async_copy(sh_outV.at[pl.ds(0, K_FG)],
                               tieV.at[pl.ds(0, K_FG)], sem),
              pltpu.async_copy(sh_outI.at[pl.ds(0, K_FG)],
                               tieI.at[pl.ds(0, K_FG)], sem),
              pltpu.async_copy(sh_outC.at[pl.ds(0, K_FG)],
                               tieC.at[pl.ds(0, K_FG)], sem)]
        for h in hs:
            h.wait()
        hs = [pltpu.async_copy(tieV.at[pl.ds(0, K_FG)],
                               sv_out.at[pl.ds(0, K_FG)], sem),
              pltpu.async_copy(tieI.at[pl.ds(0, K_FG)],
                               si_out.at[pl.ds(0, K_FG)], sem),
              pltpu.async_copy(tieC.at[pl.ds(0, K_FG)],
                               scl_out.at[pl.ds(0, K_FG)], sem)]
        for h in hs:
            h.wait()

    @pl.when((cid == 1) & (tid == 0))
    def _():
        hs = [pltpu.async_copy(sh_outV.at[pl.ds(OUT_PAD, K_BG)],
                               tieV.at[pl.ds(0, K_BG)], sem),
              pltpu.async_copy(sh_outI.at[pl.ds(OUT_PAD, K_BG)],
                               tieI.at[pl.ds(0, K_BG)], sem),
              pltpu.async_copy(sh_outC.at[pl.ds(OUT_PAD, K_BG)],
                               tieC.at[pl.ds(0, K_BG)], sem)]
        for h in hs:
            h.wait()
        hs = [pltpu.async_copy(tieV.at[pl.ds(0, K_BG)],
                               sv_out.at[pl.ds(K_FG, K_BG)], sem),
              pltpu.async_copy(tieI.at[pl.ds(0, K_BG)],
                               si_out.at[pl.ds(K_FG, K_BG)], sem),
              pltpu.async_copy(tieC.at[pl.ds(0, K_BG)],
                               scl_out.at[pl.ds(K_FG, K_BG)], sem)]
        for h in hs:
            h.wait()


@jax.jit
def kernel(proposal_boxes, gt_boxes, scores, gt_classes):
    mesh = plsc.VectorSubcoreMesh(core_axis_name="c", subcore_axis_name="s",
                                  num_cores=2, num_subcores=16)
    f32 = jnp.float32
    i32 = jnp.int32

    pb = jnp.pad(proposal_boxes, ((0, N_PAD - N_PROP), (0, 0)))
    px1, py1, px2, py2 = (pb[:, i] for i in range(4))
    sc = jnp.pad(scores, (0, N_PAD - N_PROP))
    g_rep = [jnp.repeat(gt_boxes[:, i], 16) for i in range(4)]
    gcl_rep = jnp.repeat(gt_classes, 16)

    phase_a = pl.kernel(
        _phase_a_body,
        out_type=(
            jax.ShapeDtypeStruct((N_PAD,), f32),
            jax.ShapeDtypeStruct((N_PAD,), f32),
            jax.ShapeDtypeStruct((N_PAD,), f32),
            jax.ShapeDtypeStruct((N_PAD,), i32),
        ),
        mesh=mesh,
        compiler_params=pltpu.CompilerParams(needs_layout_passes=False),
        scratch_types=[
            pltpu.VMEM((CHUNK,), f32), pltpu.VMEM((CHUNK,), f32),
            pltpu.VMEM((CHUNK,), f32), pltpu.VMEM((CHUNK,), f32),
            pltpu.VMEM((CHUNK,), f32),
            pltpu.VMEM((N_GT * 16,), f32), pltpu.VMEM((N_GT * 16,), f32),
            pltpu.VMEM((N_GT * 16,), f32), pltpu.VMEM((N_GT * 16,), f32),
            pltpu.VMEM((N_GT * 16,), i32),
            pltpu.VMEM((CHUNK,), f32), pltpu.VMEM((CHUNK,), f32),
            pltpu.VMEM((CHUNK,), f32), pltpu.VMEM((CHUNK,), i32),
            pltpu.VMEM((N_GT * 16,), f32),
        ],
    )
    iou_p, fg_key, bg_key, cls = phase_a(
        px1, py1, px2, py2, sc, g_rep[0], g_rep[1], g_rep[2], g_rep[3], gcl_rep)

    phase_b = pl.kernel(
        _phase_b_body,
        out_type=(
            jax.ShapeDtypeStruct((512,), f32),
            jax.ShapeDtypeStruct((512,), i32),
            jax.ShapeDtypeStruct((512,), i32),
        ),
        mesh=mesh,
        compiler_params=pltpu.CompilerParams(needs_layout_passes=False),
        scratch_types=[
            pltpu.VMEM((SLICE,), f32),
            pltpu.VMEM((SLICE,), jnp.uint32),
            pltpu.VMEM((SLICE,), i32),
            pltpu.VMEM((2048,), i32),
            pltpu.VMEM((2048,), i32),
            pltpu.VMEM((K_PAD,), i32),
            pltpu.VMEM((K_PAD,), i32),
            pltpu.VMEM((K_PAD,), i32),
            pltpu.VMEM((K_PAD,), i32),
            pltpu.VMEM((K_PAD,), f32),
            pltpu.VMEM((K_PAD,), i32),
            pltpu.VMEM((K_PAD,), i32),
            pltpu.VMEM((K_PAD,), i32),
            pltpu.VMEM((K_PAD,), i32),
            pltpu.VMEM((K_PAD,), i32),
            pltpu.VMEM((K_PAD,), i32),
            pltpu.VMEM((16,), f32),
            pltpu.VMEM((16,), i32),
            pltpu.VMEM((16,), i32),
            pltpu.VMEM((16,), i32),
            pltpu.VMEM((16,), f32),
            pltpu.VMEM((16,), i32),
            pltpu.VMEM((16,), i32),
            pltpu.VMEM((16,), i32),
            pltpu.VMEM((32,), i32),
            pltpu.VMEM((512,), i32),
            pltpu.VMEM_SHARED((2 * 6144,), i32),
            pltpu.VMEM_SHARED((2 * 512,), i32),
            pltpu.VMEM_SHARED((2 * K_PAD,), i32),
            pltpu.VMEM_SHARED((2 * K_PAD,), i32),
            pltpu.VMEM_SHARED((2 * K_PAD,), i32),
            pltpu.VMEM_SHARED((2 * OUT_PAD,), f32),
            pltpu.VMEM_SHARED((2 * OUT_PAD,), i32),
            pltpu.VMEM_SHARED((2 * OUT_PAD,), i32),
            pltpu.SemaphoreType.DMA,
        ],
    )
    sv, si, scl = phase_b(fg_key, bg_key, cls)

    return iou_p[:N_PROP], si, scl, sv


# speedup vs baseline: 2.4683x; 1.0502x over previous
import functools

import jax
import jax.numpy as jnp
from jax import lax
from jax.experimental import pallas as pl
from jax.experimental.pallas import tpu as pltpu
from jax.experimental.pallas import tpu_sc as plsc

N_PROP = 20000
N_PAD = 20480
N_GT = 64
NW = 32
CHUNK = N_PAD // NW
NUM_CLASSES = 80
IOU_THRESH = 0.5
K_FG = 128
K_BG = 384
K_PAD = 400
OUT_PAD = 448
NEG_INF = float("-inf")


def _iota16():
    return lax.iota(jnp.int32, 16)


def _phase_a_body(px1, py1, px2, py2, sc, gx1, gy1, gx2, gy2, gcl,
                  iou_out, fg_out, bg_out, cls_out,
                  px1_v, py1_v, px2_v, py2_v, sc_v,
                  g1_v, g2_v, g3_v, g4_v, gc_v,
                  iou_s, fg_s, bg_s, cls_s, ag_v):
    wid = lax.axis_index("s") * 2 + lax.axis_index("c")
    base = wid * CHUNK

    pltpu.sync_copy(px1.at[pl.ds(base, CHUNK)], px1_v)
    pltpu.sync_copy(py1.at[pl.ds(base, CHUNK)], py1_v)
    pltpu.sync_copy(px2.at[pl.ds(base, CHUNK)], px2_v)
    pltpu.sync_copy(py2.at[pl.ds(base, CHUNK)], py2_v)
    pltpu.sync_copy(sc.at[pl.ds(base, CHUNK)], sc_v)
    pltpu.sync_copy(gx1, g1_v)
    pltpu.sync_copy(gy1, g2_v)
    pltpu.sync_copy(gx2, g3_v)
    pltpu.sync_copy(gy2, g4_v)
    pltpu.sync_copy(gcl, gc_v)

    def area_body(g, _):
        go = g * 16
        ag_v[pl.ds(go, 16)] = ((g3_v[pl.ds(go, 16)] - g1_v[pl.ds(go, 16)])
                               * (g4_v[pl.ds(go, 16)] - g2_v[pl.ds(go, 16)]))
        return ()

    lax.fori_loop(0, N_GT, area_body, ())

    def chunk_body(v, _):
        oa = v * 32
        ob = oa + 16
        pxa1 = px1_v[pl.ds(oa, 16)]
        pya1 = py1_v[pl.ds(oa, 16)]
        pxa2 = px2_v[pl.ds(oa, 16)]
        pya2 = py2_v[pl.ds(oa, 16)]
        pxb1 = px1_v[pl.ds(ob, 16)]
        pyb1 = py1_v[pl.ds(ob, 16)]
        pxb2 = px2_v[pl.ds(ob, 16)]
        pyb2 = py2_v[pl.ds(ob, 16)]
        apa = (pxa2 - pxa1) * (pya2 - pya1)
        apb = (pxb2 - pxb1) * (pyb2 - pyb1)

        def g_body(g, c):
            bia, bca, bib, bcb = c
            go = g * 16
            g_x1 = g1_v[pl.ds(go, 16)]
            g_y1 = g2_v[pl.ds(go, 16)]
            g_x2 = g3_v[pl.ds(go, 16)]
            g_y2 = g4_v[pl.ds(go, 16)]
            g_c = gc_v[pl.ds(go, 16)]
            ag = ag_v[pl.ds(go, 16)]
            wa = jnp.maximum(
                jnp.minimum(g_x2, pxa2) - jnp.maximum(g_x1, pxa1), 0.0)
            ha = jnp.maximum(
                jnp.minimum(g_y2, pya2) - jnp.maximum(g_y1, pya1), 0.0)
            ia = wa * ha
            ioua = ia / jnp.maximum(ag + apa - ia, 1e-9)
            upda = ioua > bia
            wb = jnp.maximum(
                jnp.minimum(g_x2, pxb2) - jnp.maximum(g_x1, pxb1), 0.0)
            hb = jnp.maximum(
                jnp.minimum(g_y2, pyb2) - jnp.maximum(g_y1, pyb1), 0.0)
            ib = wb * hb
            ioub = ib / jnp.maximum(ag + apb - ib, 1e-9)
            updb = ioub > bib
            return (jnp.where(upda, ioua, bia), jnp.where(upda, g_c, bca),
                    jnp.where(updb, ioub, bib), jnp.where(updb, g_c, bcb))

        bi0 = jnp.full((16,), -1.0, jnp.float32)
        bc0 = jnp.zeros((16,), jnp.int32)
        bia, bca, bib, bcb = lax.fori_loop(0, N_GT, g_body,
                                           (bi0, bc0, bi0, bc0))

        neg = jnp.full((16,), NEG_INF, jnp.float32)
        for o, bi, bc in ((oa, bia, bca), (ob, bib, bcb)):
            s = sc_v[pl.ds(o, 16)]
            gidx = base + o + _iota16()
            valid = gidx < N_PROP
            matched = bi >= IOU_THRESH
            fg = jnp.where(
                valid,
                jnp.where(matched, bi, jnp.full((16,), -1.0, jnp.float32)),
                neg)
            bg = jnp.where(
                valid,
                jnp.where(matched, jnp.full((16,), -1e9, jnp.float32), s),
                neg)
            cl = jnp.where(matched, bc, jnp.full((16,), NUM_CLASSES,
                                                 jnp.int32))
            iou_s[pl.ds(o, 16)] = bi
            fg_s[pl.ds(o, 16)] = fg
            bg_s[pl.ds(o, 16)] = bg
            cls_s[pl.ds(o, 16)] = cl
        return ()

    lax.fori_loop(0, CHUNK // 32, chunk_body, ())

    pltpu.sync_copy(iou_s, iou_out.at[pl.ds(base, CHUNK)])
    pltpu.sync_copy(fg_s, fg_out.at[pl.ds(base, CHUNK)])
    pltpu.sync_copy(bg_s, bg_out.at[pl.ds(base, CHUNK)])
    pltpu.sync_copy(cls_s, cls_out.at[pl.ds(base, CHUNK)])


SLICE = N_PAD // 16
NVB = SLICE // 16
INT_MIN = -2147483648
INT_MAX = 2147483647


def _phase_b_body(fg_hbm, bg_hbm, cls_hbm, sv_out, si_out, scl_out,
                  key_v, u_v, cls_v, hist, hidx,
                  candU, candI, candC, candD,
                  tieV, tieI, tieC, tieD,
                  dU, dI, dC, st16a, st16b, st16c, st16d,
                  st16e, st16f, st16g, st16h, cnt32, cntb,
                  sh_hist, sh_cnt, sh_candU, sh_candI, sh_candC,
                  sh_outV, sh_outI, sh_outC, sem):
    cid = lax.axis_index("c")
    tid = lax.axis_index("s")
    k = jnp.where(cid == 0, K_FG, K_BG)
    kvec = jnp.broadcast_to(k, (16,))
    one = jnp.ones((16,), jnp.int32)
    zero = jnp.zeros((16,), jnp.int32)
    iot = _iota16()
    sgn = jnp.int32(INT_MIN)
    base = tid * SLICE

    @pl.when(cid == 0)
    def _():
        h1 = pltpu.async_copy(fg_hbm.at[pl.ds(base, SLICE)], key_v, sem)
        h2 = pltpu.async_copy(cls_hbm.at[pl.ds(base, SLICE)], cls_v, sem)
        h1.wait()
        h2.wait()

    @pl.when(cid == 1)
    def _():
        h1 = pltpu.async_copy(bg_hbm.at[pl.ds(base, SLICE)], key_v, sem)
        h2 = pltpu.async_copy(cls_hbm.at[pl.ds(base, SLICE)], cls_v, sem)
        h1.wait()
        h2.wait()

    def z0_body(v, _):
        hist[pl.ds(v * 16, 16)] = zero
        return ()

    lax.fori_loop(0, 128, z0_body, ())
    hbase = cid * 6144
    hz = pltpu.async_copy(hist.at[pl.ds(0, 384)],
                          sh_hist.at[pl.ds(hbase + tid * 384, 384)], sem)

    def u_body(v, _):
        o = v * 16
        b = plsc.bitcast(key_v[pl.ds(o, 16)], jnp.int32)
        u = jnp.where(b < 0, ~b, b ^ sgn)
        u_v[pl.ds(o, 16)] = plsc.bitcast(u, jnp.uint32)
        return ()

    lax.fori_loop(0, NVB, u_body, ())
    hz.wait()
    plsc.subcore_barrier()

    pre = jnp.uint32(0)
    rem = k
    for pnum, (shift, width) in enumerate(((21, 11), (10, 11), (0, 10))):
        nbins = 1 << width
        nb = nbins // 16
        pbase = hbase + pnum * 2048

        def z_body(v, _, pbase=pbase):
            hidx[pl.ds(v * 16, 16)] = pbase + v * 16 + iot
            if pnum:
                hist[pl.ds(v * 16, 16)] = zero
            return ()

        lax.fori_loop(0, 128, z_body, ())

        dmask = jnp.uint32(nbins - 1)
        hi = shift + width
        pre_hi = lax.shift_right_logical(pre, jnp.uint32(hi)) if hi < 32 else None

        def acc_body(v, _, shift=shift, hi=hi, dmask=dmask, pre_hi=pre_hi):
            u = u_v[pl.ds(v * 16, 16)]
            dig = lax.shift_right_logical(u, jnp.uint32(shift)) & dmask
            if pre_hi is None:
                inc = one
            else:
                uh = lax.shift_right_logical(u, jnp.uint32(hi))
                inc = jnp.where(uh == jnp.broadcast_to(pre_hi, (16,)), one, zero)
            plsc.addupdate_scatter(hist, [plsc.bitcast(dig, jnp.int32)], inc)
            return ()

        lax.fori_loop(0, NVB, acc_body, ())

        pltpu.sync_copy(hist, sh_hist.at[hidx], add=True)
        plsc.subcore_barrier()
        pltpu.sync_copy(sh_hist.at[pl.ds(pbase, 2048)], hist.at[pl.ds(0, 2048)])

        def s_cond(c, nb=nb):
            j, above, found, d, g = c
            return (found == 0) & (j < nb)

        def s_step(c, nb=nb):
            j, above, found, d, g = c
            v = nb - 1 - j
            h = hist[pl.ds(v * 16, 16)]
            s_ge = lax.rev(plsc.cumsum(lax.rev(h, (0,))), (0,))
            tot = above + s_ge
            npos = jnp.max(
                jnp.where(tot >= jnp.broadcast_to(rem, (16,)), one, zero)
                * (iot + 1))
            lv = jnp.broadcast_to(npos - 1, (16,))
            gv = above + jnp.sum(jnp.where(iot > lv, h, zero))
            dv = v * 16 + npos - 1
            d = jnp.where(npos > 0, dv, d)
            g = jnp.where(npos > 0, gv, g)
            found = jnp.where(npos > 0, 1, 0)
            above = above + jnp.sum(h)
            return (j + 1, above, found, d, g)

        _, _, _, d, g = lax.while_loop(
            s_cond, s_step,
            (jnp.int32(0), jnp.int32(0), jnp.int32(0), jnp.int32(0),
             jnp.int32(0)))
        pre = pre | lax.shift_left(
            lax.bitcast_convert_type(d, jnp.uint32), jnp.uint32(shift))
        rem = rem - g

    t_u = pre
    tv = jnp.broadcast_to(t_u, (16,))

    def c_body(v, c):
        ng, ne = c
        u = u_v[pl.ds(v * 16, 16)]
        ng = ng + jnp.sum(jnp.where(u > tv, one, zero))
        ne = ne + jnp.sum(jnp.where(u == tv, one, zero))
        return (ng, ne)

    ng_me, ne_me = lax.fori_loop(0, NVB, c_body,
                                 (jnp.int32(0), jnp.int32(0)))
    cbase = cid * 512
    cnt32[pl.ds(0, 16)] = jnp.broadcast_to(ng_me, (16,))
    cnt32[pl.ds(16, 16)] = jnp.broadcast_to(ne_me, (16,))
    hc = pltpu.async_copy(cnt32, sh_cnt.at[pl.ds(cbase + tid * 32, 32)], sem)

    @pl.when(tid == 0)
    def _():
        def pad_body(v, _):
            o = v * 16
            dU[pl.ds(o, 16)] = jnp.full((16,), INT_MIN, jnp.int32)
            dI[pl.ds(o, 16)] = jnp.full((16,), INT_MAX, jnp.int32)
            return ()

        lax.fori_loop(0, K_PAD // 16, pad_body, ())
        p1 = pltpu.async_copy(dU, sh_candU.at[pl.ds(cid * K_PAD, K_PAD)], sem)
        p2 = pltpu.async_copy(dI, sh_candI.at[pl.ds(cid * K_PAD, K_PAD)], sem)
        p1.wait()
        p2.wait()

    hc.wait()
    plsc.subcore_barrier()

    pltpu.sync_copy(sh_cnt.at[pl.ds(cbase, 512)], cntb)
    g_cnt = plsc.load_gather(cntb, [iot * 33])
    e_cnt = plsc.load_gather(cntb, [iot * 33 + 16])
    m_tot = jnp.sum(g_cnt)
    ex_g = plsc.cumsum(g_cnt) - g_cnt
    ex_e = plsc.cumsum(e_cnt) - e_cnt
    tsel = jnp.where(iot == jnp.broadcast_to(tid, (16,)), one, zero)
    base_gt = jnp.sum(tsel * ex_g)
    base_eq = jnp.sum(tsel * ex_e)
    mvec = jnp.broadcast_to(m_tot, (16,))

    junk_c = jnp.broadcast_to(cid * K_PAD + K_BG + tid, (16,))
    junk_o = jnp.broadcast_to(cid * OUT_PAD + K_BG + tid, (16,))

    def pf_body(v, _):
        o = v * 16
        candD[pl.ds(o, 16)] = junk_c
        tieD[pl.ds(o, 16)] = junk_o
        return ()

    lax.fori_loop(0, K_PAD // 16, pf_body, ())

    def comp_body(v, carry):
        off, toff, eqc = carry
        o = v * 16
        u = u_v[pl.ds(o, 16)]
        any_rel = jnp.sum(jnp.where(u >= tv, one, zero))

        def do(carry):
            off, toff, eqc = carry
            kv = key_v[pl.ds(o, 16)]
            cv = cls_v[pl.ds(o, 16)]
            gidx = base + o + iot
            m_gt = u > tv
            s = plsc.bitcast(u, jnp.int32) ^ sgn
            dcand = cid * K_PAD + base_gt + off + plsc.cumsum(
                jnp.where(m_gt, one, zero)) - 1
            plsc.store_compressed(candU.at[pl.ds(off, 16)], s, mask=m_gt)
            plsc.store_compressed(candI.at[pl.ds(off, 16)], gidx, mask=m_gt)
            plsc.store_compressed(candC.at[pl.ds(off, 16)], cv, mask=m_gt)
            plsc.store_compressed(candD.at[pl.ds(off, 16)], dcand, mask=m_gt)
            n_gt = jnp.sum(jnp.where(m_gt, one, zero))
            m_eq = u == tv
            eq1 = jnp.where(m_eq, one, zero)
            pos = eqc + plsc.cumsum(eq1) - 1
            dest = mvec + jnp.broadcast_to(base_eq, (16,)) + pos
            keep = jnp.where(m_eq, dest, kvec) < kvec
            plsc.store_compressed(tieV.at[pl.ds(toff, 16)], kv, mask=keep)
            plsc.store_compressed(tieI.at[pl.ds(toff, 16)], gidx, mask=keep)
            plsc.store_compressed(tieC.at[pl.ds(toff, 16)], cv, mask=keep)
            plsc.store_compressed(tieD.at[pl.ds(toff, 16)],
                                  cid * OUT_PAD + dest, mask=keep)
            return (off + n_gt, toff + jnp.sum(jnp.where(keep, one, zero)),
                    eqc + jnp.sum(eq1))

        return lax.cond(any_rel > 0, do, lambda c: c, (off, toff, eqc))

    lax.fori_loop(0, NVB, comp_body,
                  (jnp.int32(0), jnp.int32(0), jnp.int32(0)))

    hs = [pltpu.async_copy(candU, sh_candU.at[candD], sem),
          pltpu.async_copy(candI, sh_candI.at[candD], sem),
          pltpu.async_copy(candC, sh_candC.at[candD], sem),
          pltpu.async_copy(tieV, sh_outV.at[tieD], sem),
          pltpu.async_copy(tieI, sh_outI.at[tieD], sem),
          pltpu.async_copy(tieC, sh_outC.at[tieD], sem)]
    for h in hs:
        h.wait()
    plsc.subcore_barrier()

    hs = [pltpu.async_copy(sh_candU.at[pl.ds(cid * K_PAD, K_PAD)], dU, sem),
          pltpu.async_copy(sh_candI.at[pl.ds(cid * K_PAD, K_PAD)], dI, sem),
          pltpu.async_copy(sh_candC.at[pl.ds(cid * K_PAD, K_PAD)], dC, sem)]
    for h in hs:
        h.wait()
    dU[pl.ds(K_BG, 16)] = jnp.full((16,), INT_MIN, jnp.int32)
    dI[pl.ds(K_BG, 16)] = jnp.full((16,), INT_MAX, jnp.int32)

    def rank_block(bb, sta, stb, stc, std):
        ao = bb * 16
        aU = dU[pl.ds(ao, 16)]
        aI = dI[pl.ds(ao, 16)]
        aC = dC[pl.ds(ao, 16)]

        def b_body(j, acc):
            def r_body(r, acc):
                idx = j * 16 + ((iot + r) & 15)
                bU = plsc.load_gather(dU, [idx])
                bI = plsc.load_gather(dI, [idx])
                tie = jnp.where(bI < aI, one, zero)
                better = jnp.where(bU > aU, one,
                                   jnp.where(bU == aU, tie, zero))
                return acc + better

            return lax.fori_loop(0, 16, r_body, acc)

        rank = lax.fori_loop(0, K_PAD // 16, b_body, zero)
        lanepos = ao + iot
        wmask = jnp.where(lanepos < mvec, rank, kvec) < kvec
        rc = jnp.maximum(jnp.minimum(rank, kvec - 1), 0)
        ui = aU ^ sgn
        vbits = jnp.where(aU >= 0, aU, ~ui)
        sta[...] = plsc.bitcast(vbits, jnp.float32)
        stb[...] = aI
        stc[...] = aC
        std[...] = jnp.where(wmask, cid * OUT_PAD + rc, junk_o)
        hs = [pltpu.async_copy(sta, sh_outV.at[std], sem),
              pltpu.async_copy(stb, sh_outI.at[std], sem),
              pltpu.async_copy(stc, sh_outC.at[std], sem)]
        for h in hs:
            h.wait()

    rank_block(tid, st16a, st16b, st16c, st16d)

    @pl.when(tid < (K_PAD // 16) - 16)
    def _():
        rank_block(tid + 16, st16e, st16f, st16g, st16h)

    plsc.subcore_barrier()

    @pl.when((cid == 0) & (tid == 0))
    def _():
        hs = [pltpu.async_copy(sh_outV.at[pl.ds(0, K_FG)],
                               tieV.at[pl.ds(0, K_FG)], sem),
              pltpu.async_copy(sh_outI.at[pl.ds(0, K_FG)],
                               tieI.at[pl.ds(0, K_FG)], sem),
              pltpu.async_copy(sh_outC.at[pl.ds(0, K_FG)],
                               tieC.at[pl.ds(0, K_FG)], sem)]
        for h in hs:
            h.wait()
        hs = [pltpu.async_copy(tieV.at[pl.ds(0, K_FG)],
                               sv_out.at[pl.ds(0, K_FG)], sem),
              pltpu.async_copy(tieI.at[pl.ds(0, K_FG)],
                               si_out.at[pl.ds(0, K_FG)], sem),
              pltpu.async_copy(tieC.at[pl.ds(0, K_FG)],
                               scl_out.at[pl.ds(0, K_FG)], sem)]
        for h in hs:
            h.wait()

    @pl.when((cid == 1) & (tid == 0))
    def _():
        hs = [pltpu.async_copy(sh_outV.at[pl.ds(OUT_PAD, K_BG)],
                               tieV.at[pl.ds(0, K_BG)], sem),
              pltpu.async_copy(sh_outI.at[pl.ds(OUT_PAD, K_BG)],
                               tieI.at[pl.ds(0, K_BG)], sem),
              pltpu.async_copy(sh_outC.at[pl.ds(OUT_PAD, K_BG)],
                               tieC.at[pl.ds(0, K_BG)], sem)]
        for h in hs:
            h.wait()
        hs = [pltpu.async_copy(tieV.at[pl.ds(0, K_BG)],
                               sv_out.at[pl.ds(K_FG, K_BG)], sem),
              pltpu.async_copy(tieI.at[pl.ds(0, K_BG)],
                               si_out.at[pl.ds(K_FG, K_BG)], sem),
              pltpu.async_copy(tieC.at[pl.ds(0, K_BG)],
                               scl_out.at[pl.ds(K_FG, K_BG)], sem)]
        for h in hs:
            h.wait()


@jax.jit
def kernel(proposal_boxes, gt_boxes, scores, gt_classes):
    mesh = plsc.VectorSubcoreMesh(core_axis_name="c", subcore_axis_name="s",
                                  num_cores=2, num_subcores=16)
    f32 = jnp.float32
    i32 = jnp.int32

    pb = jnp.pad(proposal_boxes, ((0, N_PAD - N_PROP), (0, 0)))
    px1, py1, px2, py2 = (pb[:, i] for i in range(4))
    sc = jnp.pad(scores, (0, N_PAD - N_PROP))
    g_rep = [jnp.repeat(gt_boxes[:, i], 16) for i in range(4)]
    gcl_rep = jnp.repeat(gt_classes, 16)

    phase_a = pl.kernel(
        _phase_a_body,
        out_type=(
            jax.ShapeDtypeStruct((N_PAD,), f32),
            jax.ShapeDtypeStruct((N_PAD,), f32),
            jax.ShapeDtypeStruct((N_PAD,), f32),
            jax.ShapeDtypeStruct((N_PAD,), i32),
        ),
        mesh=mesh,
        compiler_params=pltpu.CompilerParams(needs_layout_passes=False),
        scratch_types=[
            pltpu.VMEM((CHUNK,), f32), pltpu.VMEM((CHUNK,), f32),
            pltpu.VMEM((CHUNK,), f32), pltpu.VMEM((CHUNK,), f32),
            pltpu.VMEM((CHUNK,), f32),
            pltpu.VMEM((N_GT * 16,), f32), pltpu.VMEM((N_GT * 16,), f32),
            pltpu.VMEM((N_GT * 16,), f32), pltpu.VMEM((N_GT * 16,), f32),
            pltpu.VMEM((N_GT * 16,), i32),
            pltpu.VMEM((CHUNK,), f32), pltpu.VMEM((CHUNK,), f32),
            pltpu.VMEM((CHUNK,), f32), pltpu.VMEM((CHUNK,), i32),
            pltpu.VMEM((N_GT * 16,), f32),
        ],
    )
    iou_p, fg_key, bg_key, cls = phase_a(
        px1, py1, px2, py2, sc, g_rep[0], g_rep[1], g_rep[2], g_rep[3], gcl_rep)

    phase_b = pl.kernel(
        _phase_b_body,
        out_type=(
            jax.ShapeDtypeStruct((512,), f32),
            jax.ShapeDtypeStruct((512,), i32),
            jax.ShapeDtypeStruct((512,), i32),
        ),
        mesh=mesh,
        compiler_params=pltpu.CompilerParams(needs_layout_passes=False),
        scratch_types=[
            pltpu.VMEM((SLICE,), f32),
            pltpu.VMEM((SLICE,), jnp.uint32),
            pltpu.VMEM((SLICE,), i32),
            pltpu.VMEM((2048,), i32),
            pltpu.VMEM((2048,), i32),
            pltpu.VMEM((K_PAD,), i32),
            pltpu.VMEM((K_PAD,), i32),
            pltpu.VMEM((K_PAD,), i32),
            pltpu.VMEM((K_PAD,), i32),
            pltpu.VMEM((K_PAD,), f32),
            pltpu.VMEM((K_PAD,), i32),
            pltpu.VMEM((K_PAD,), i32),
            pltpu.VMEM((K_PAD,), i32),
            pltpu.VMEM((K_PAD,), i32),
            pltpu.VMEM((K_PAD,), i32),
            pltpu.VMEM((K_PAD,), i32),
            pltpu.VMEM((16,), f32),
            pltpu.VMEM((16,), i32),
            pltpu.VMEM((16,), i32),
            pltpu.VMEM((16,), i32),
            pltpu.VMEM((16,), f32),
            pltpu.VMEM((16,), i32),
            pltpu.VMEM((16,), i32),
            pltpu.VMEM((16,), i32),
            pltpu.VMEM((32,), i32),
            pltpu.VMEM((512,), i32),
            pltpu.VMEM_SHARED((2 * 6144,), i32),
            pltpu.VMEM_SHARED((2 * 512,), i32),
            pltpu.VMEM_SHARED((2 * K_PAD,), i32),
            pltpu.VMEM_SHARED((2 * K_PAD,), i32),
            pltpu.VMEM_SHARED((2 * K_PAD,), i32),
            pltpu.VMEM_SHARED((2 * OUT_PAD,), f32),
            pltpu.VMEM_SHARED((2 * OUT_PAD,), i32),
            pltpu.VMEM_SHARED((2 * OUT_PAD,), i32),
            pltpu.SemaphoreType.DMA,
        ],
    )
    sv, si, scl = phase_b(fg_key, bg_key, cls)

    return iou_p[:N_PROP], si, scl, sv


# speedup vs baseline: 2.4877x; 1.0078x over previous
import functools

import jax
import jax.numpy as jnp
from jax import lax
from jax.experimental import pallas as pl
from jax.experimental.pallas import tpu as pltpu
from jax.experimental.pallas import tpu_sc as plsc

N_PROP = 20000
N_PAD = 20480
N_GT = 64
NW = 32
CHUNK = N_PAD // NW
NUM_CLASSES = 80
IOU_THRESH = 0.5
K_FG = 128
K_BG = 384
K_PAD = 400
OUT_PAD = 448
NEG_INF = float("-inf")


def _iota16():
    return lax.iota(jnp.int32, 16)


def _phase_a_body(px1, py1, px2, py2, sc, gx1, gy1, gx2, gy2, gcl,
                  iou_out, fg_out, bg_out, cls_out,
                  px1_v, py1_v, px2_v, py2_v, sc_v,
                  g1_v, g2_v, g3_v, g4_v, gc_v,
                  iou_s, fg_s, bg_s, cls_s, ag_v):
    wid = lax.axis_index("s") * 2 + lax.axis_index("c")
    base = wid * CHUNK

    pltpu.sync_copy(px1.at[pl.ds(base, CHUNK)], px1_v)
    pltpu.sync_copy(py1.at[pl.ds(base, CHUNK)], py1_v)
    pltpu.sync_copy(px2.at[pl.ds(base, CHUNK)], px2_v)
    pltpu.sync_copy(py2.at[pl.ds(base, CHUNK)], py2_v)
    pltpu.sync_copy(sc.at[pl.ds(base, CHUNK)], sc_v)
    pltpu.sync_copy(gx1, g1_v)
    pltpu.sync_copy(gy1, g2_v)
    pltpu.sync_copy(gx2, g3_v)
    pltpu.sync_copy(gy2, g4_v)
    pltpu.sync_copy(gcl, gc_v)

    def area_body(g, _):
        go = g * 16
        ag_v[pl.ds(go, 16)] = ((g3_v[pl.ds(go, 16)] - g1_v[pl.ds(go, 16)])
                               * (g4_v[pl.ds(go, 16)] - g2_v[pl.ds(go, 16)]))
        return ()

    lax.fori_loop(0, N_GT, area_body, ())

    UNR = 4

    def chunk_body(v, _):
        o0 = v * (16 * UNR)
        px1s = [px1_v[pl.ds(o0 + u * 16, 16)] for u in range(UNR)]
        py1s = [py1_v[pl.ds(o0 + u * 16, 16)] for u in range(UNR)]
        px2s = [px2_v[pl.ds(o0 + u * 16, 16)] for u in range(UNR)]
        py2s = [py2_v[pl.ds(o0 + u * 16, 16)] for u in range(UNR)]
        aps = [(px2s[u] - px1s[u]) * (py2s[u] - py1s[u]) for u in range(UNR)]

        def g_body(g, c):
            bis = list(c[:UNR])
            bcs = list(c[UNR:])
            go = g * 16
            g_x1 = g1_v[pl.ds(go, 16)]
            g_y1 = g2_v[pl.ds(go, 16)]
            g_x2 = g3_v[pl.ds(go, 16)]
            g_y2 = g4_v[pl.ds(go, 16)]
            g_c = gc_v[pl.ds(go, 16)]
            ag = ag_v[pl.ds(go, 16)]
            for u in range(UNR):
                w = jnp.maximum(
                    jnp.minimum(g_x2, px2s[u]) - jnp.maximum(g_x1, px1s[u]),
                    0.0)
                h = jnp.maximum(
                    jnp.minimum(g_y2, py2s[u]) - jnp.maximum(g_y1, py1s[u]),
                    0.0)
                inter = w * h
                iou = inter / jnp.maximum(ag + aps[u] - inter, 1e-9)
                upd = iou > bis[u]
                bis[u] = jnp.where(upd, iou, bis[u])
                bcs[u] = jnp.where(upd, g_c, bcs[u])
            return tuple(bis) + tuple(bcs)

        bi0 = jnp.full((16,), -1.0, jnp.float32)
        bc0 = jnp.zeros((16,), jnp.int32)
        res = lax.fori_loop(0, N_GT, g_body, (bi0,) * UNR + (bc0,) * UNR)

        neg = jnp.full((16,), NEG_INF, jnp.float32)
        for u in range(UNR):
            o, bi, bc = o0 + u * 16, res[u], res[UNR + u]
            s = sc_v[pl.ds(o, 16)]
            gidx = base + o + _iota16()
            valid = gidx < N_PROP
            matched = bi >= IOU_THRESH
            fg = jnp.where(
                valid,
                jnp.where(matched, bi, jnp.full((16,), -1.0, jnp.float32)),
                neg)
            bg = jnp.where(
                valid,
                jnp.where(matched, jnp.full((16,), -1e9, jnp.float32), s),
                neg)
            cl = jnp.where(matched, bc, jnp.full((16,), NUM_CLASSES,
                                                 jnp.int32))
            iou_s[pl.ds(o, 16)] = bi
            fg_s[pl.ds(o, 16)] = fg
            bg_s[pl.ds(o, 16)] = bg
            cls_s[pl.ds(o, 16)] = cl
        return ()

    lax.fori_loop(0, CHUNK // (16 * UNR), chunk_body, ())

    pltpu.sync_copy(iou_s, iou_out.at[pl.ds(base, CHUNK)])
    pltpu.sync_copy(fg_s, fg_out.at[pl.ds(base, CHUNK)])
    pltpu.sync_copy(bg_s, bg_out.at[pl.ds(base, CHUNK)])
    pltpu.sync_copy(cls_s, cls_out.at[pl.ds(base, CHUNK)])


SLICE = N_PAD // 16
NVB = SLICE // 16
INT_MIN = -2147483648
INT_MAX = 2147483647


def _phase_b_body(fg_hbm, bg_hbm, cls_hbm, sv_out, si_out, scl_out,
                  key_v, u_v, cls_v, hist, hidx,
                  candU, candI, candC, candD,
                  tieV, tieI, tieC, tieD,
                  dU, dI, dC, st16a, st16b, st16c, st16d,
                  st16e, st16f, st16g, st16h, cnt32, cntb,
                  sh_hist, sh_cnt, sh_candU, sh_candI, sh_candC,
                  sh_outV, sh_outI, sh_outC, sem):
    cid = lax.axis_index("c")
    tid = lax.axis_index("s")
    k = jnp.where(cid == 0, K_FG, K_BG)
    kvec = jnp.broadcast_to(k, (16,))
    one = jnp.ones((16,), jnp.int32)
    zero = jnp.zeros((16,), jnp.int32)
    iot = _iota16()
    sgn = jnp.int32(INT_MIN)
    base = tid * SLICE

    @pl.when(cid == 0)
    def _():
        h1 = pltpu.async_copy(fg_hbm.at[pl.ds(base, SLICE)], key_v, sem)
        h2 = pltpu.async_copy(cls_hbm.at[pl.ds(base, SLICE)], cls_v, sem)
        h1.wait()
        h2.wait()

    @pl.when(cid == 1)
    def _():
        h1 = pltpu.async_copy(bg_hbm.at[pl.ds(base, SLICE)], key_v, sem)
        h2 = pltpu.async_copy(cls_hbm.at[pl.ds(base, SLICE)], cls_v, sem)
        h1.wait()
        h2.wait()

    def z0_body(v, _):
        hist[pl.ds(v * 16, 16)] = zero
        return ()

    lax.fori_loop(0, 128, z0_body, ())
    hbase = cid * 6144
    hz = pltpu.async_copy(hist.at[pl.ds(0, 384)],
                          sh_hist.at[pl.ds(hbase + tid * 384, 384)], sem)

    def u_body(v, _):
        o = v * 16
        b = plsc.bitcast(key_v[pl.ds(o, 16)], jnp.int32)
        u = jnp.where(b < 0, ~b, b ^ sgn)
        u_v[pl.ds(o, 16)] = plsc.bitcast(u, jnp.uint32)
        return ()

    lax.fori_loop(0, NVB, u_body, ())
    hz.wait()
    plsc.subcore_barrier()

    pre = jnp.uint32(0)
    rem = k
    for pnum, (shift, width) in enumerate(((21, 11), (10, 11), (0, 10))):
        nbins = 1 << width
        nb = nbins // 16
        pbase = hbase + pnum * 2048

        def z_body(v, _, pbase=pbase):
            hidx[pl.ds(v * 16, 16)] = pbase + v * 16 + iot
            if pnum:
                hist[pl.ds(v * 16, 16)] = zero
            return ()

        lax.fori_loop(0, 128, z_body, ())

        dmask = jnp.uint32(nbins - 1)
        hi = shift + width
        pre_hi = lax.shift_right_logical(pre, jnp.uint32(hi)) if hi < 32 else None

        def acc_body(v, _, shift=shift, hi=hi, dmask=dmask, pre_hi=pre_hi):
            u = u_v[pl.ds(v * 16, 16)]
            dig = lax.shift_right_logical(u, jnp.uint32(shift)) & dmask
            if pre_hi is None:
                inc = one
            else:
                uh = lax.shift_right_logical(u, jnp.uint32(hi))
                inc = jnp.where(uh == jnp.broadcast_to(pre_hi, (16,)), one, zero)
            plsc.addupdate_scatter(hist, [plsc.bitcast(dig, jnp.int32)], inc)
            return ()

        lax.fori_loop(0, NVB, acc_body, ())

        pltpu.sync_copy(hist, sh_hist.at[hidx], add=True)
        plsc.subcore_barrier()
        pltpu.sync_copy(sh_hist.at[pl.ds(pbase, 2048)], hist.at[pl.ds(0, 2048)])

        def s_cond(c, nb=nb):
            j, above, found, d, g = c
            return (found == 0) & (j < nb)

        def s_step(c, nb=nb):
            j, above, found, d, g = c
            v = nb - 1 - j
            h = hist[pl.ds(v * 16, 16)]
            s_ge = lax.rev(plsc.cumsum(lax.rev(h, (0,))), (0,))
            tot = above + s_ge
            npos = jnp.max(
                jnp.where(tot >= jnp.broadcast_to(rem, (16,)), one, zero)
                * (iot + 1))
            lv = jnp.broadcast_to(npos - 1, (16,))
            gv = above + jnp.sum(jnp.where(iot > lv, h, zero))
            dv = v * 16 + npos - 1
            d = jnp.where(npos > 0, dv, d)
            g = jnp.where(npos > 0, gv, g)
            found = jnp.where(npos > 0, 1, 0)
            above = above + jnp.sum(h)
            return (j + 1, above, found, d, g)

        _, _, _, d, g = lax.while_loop(
            s_cond, s_step,
            (jnp.int32(0), jnp.int32(0), jnp.int32(0), jnp.int32(0),
             jnp.int32(0)))
        pre = pre | lax.shift_left(
            lax.bitcast_convert_type(d, jnp.uint32), jnp.uint32(shift))
        rem = rem - g

    t_u = pre
    tv = jnp.broadcast_to(t_u, (16,))

    def c_body(v, c):
        ng, ne = c
        u = u_v[pl.ds(v * 16, 16)]
        ng = ng + jnp.sum(jnp.where(u > tv, one, zero))
        ne = ne + jnp.sum(jnp.where(u == tv, one, zero))
        return (ng, ne)

    ng_me, ne_me = lax.fori_loop(0, NVB, c_body,
                                 (jnp.int32(0), jnp.int32(0)))
    cbase = cid * 512
    cnt32[pl.ds(0, 16)] = jnp.broadcast_to(ng_me, (16,))
    cnt32[pl.ds(16, 16)] = jnp.broadcast_to(ne_me, (16,))
    hc = pltpu.async_copy(cnt32, sh_cnt.at[pl.ds(cbase + tid * 32, 32)], sem)

    @pl.when(tid == 0)
    def _():
        def pad_body(v, _):
            o = v * 16
            dU[pl.ds(o, 16)] = jnp.full((16,), INT_MIN, jnp.int32)
            dI[pl.ds(o, 16)] = jnp.full((16,), INT_MAX, jnp.int32)
            return ()

        lax.fori_loop(0, K_PAD // 16, pad_body, ())
        p1 = pltpu.async_copy(dU, sh_candU.at[pl.ds(cid * K_PAD, K_PAD)], sem)
        p2 = pltpu.async_copy(dI, sh_candI.at[pl.ds(cid * K_PAD, K_PAD)], sem)
        p1.wait()
        p2.wait()

    hc.wait()
    plsc.subcore_barrier()

    pltpu.sync_copy(sh_cnt.at[pl.ds(cbase, 512)], cntb)
    g_cnt = plsc.load_gather(cntb, [iot * 33])
    e_cnt = plsc.load_gather(cntb, [iot * 33 + 16])
    m_tot = jnp.sum(g_cnt)
    ex_g = plsc.cumsum(g_cnt) - g_cnt
    ex_e = plsc.cumsum(e_cnt) - e_cnt
    tsel = jnp.where(iot == jnp.broadcast_to(tid, (16,)), one, zero)
    base_gt = jnp.sum(tsel * ex_g)
    base_eq = jnp.sum(tsel * ex_e)
    mvec = jnp.broadcast_to(m_tot, (16,))

    junk_c = jnp.broadcast_to(cid * K_PAD + K_BG + tid, (16,))
    junk_o = jnp.broadcast_to(cid * OUT_PAD + K_BG + tid, (16,))

    def pf_body(v, _):
        o = v * 16
        candD[pl.ds(o, 16)] = junk_c
        tieD[pl.ds(o, 16)] = junk_o
        return ()

    lax.fori_loop(0, K_PAD // 16, pf_body, ())

    def comp_body(v, carry):
        off, toff, eqc = carry
        o = v * 16
        u = u_v[pl.ds(o, 16)]
        any_rel = jnp.sum(jnp.where(u >= tv, one, zero))

        def do(carry):
            off, toff, eqc = carry
            kv = key_v[pl.ds(o, 16)]
            cv = cls_v[pl.ds(o, 16)]
            gidx = base + o + iot
            m_gt = u > tv
            s = plsc.bitcast(u, jnp.int32) ^ sgn
            dcand = cid * K_PAD + base_gt + off + plsc.cumsum(
                jnp.where(m_gt, one, zero)) - 1
            plsc.store_compressed(candU.at[pl.ds(off, 16)], s, mask=m_gt)
            plsc.store_compressed(candI.at[pl.ds(off, 16)], gidx, mask=m_gt)
            plsc.store_compressed(candC.at[pl.ds(off, 16)], cv, mask=m_gt)
            plsc.store_compressed(candD.at[pl.ds(off, 16)], dcand, mask=m_gt)
            n_gt = jnp.sum(jnp.where(m_gt, one, zero))
            m_eq = u == tv
            eq1 = jnp.where(m_eq, one, zero)
            pos = eqc + plsc.cumsum(eq1) - 1
            dest = mvec + jnp.broadcast_to(base_eq, (16,)) + pos
            keep = jnp.where(m_eq, dest, kvec) < kvec
            plsc.store_compressed(tieV.at[pl.ds(toff, 16)], kv, mask=keep)
            plsc.store_compressed(tieI.at[pl.ds(toff, 16)], gidx, mask=keep)
            plsc.store_compressed(tieC.at[pl.ds(toff, 16)], cv, mask=keep)
            plsc.store_compressed(tieD.at[pl.ds(toff, 16)],
                                  cid * OUT_PAD + dest, mask=keep)
            return (off + n_gt, toff + jnp.sum(jnp.where(keep, one, zero)),
                    eqc + jnp.sum(eq1))

        return lax.cond(any_rel > 0, do, lambda c: c, (off, toff, eqc))

    lax.fori_loop(0, NVB, comp_body,
                  (jnp.int32(0), jnp.int32(0), jnp.int32(0)))

    hs = [pltpu.async_copy(candU, sh_candU.at[candD], sem),
          pltpu.async_copy(candI, sh_candI.at[candD], sem),
          pltpu.async_copy(candC, sh_candC.at[candD], sem),
          pltpu.async_copy(tieV, sh_outV.at[tieD], sem),
          pltpu.async_copy(tieI, sh_outI.at[tieD], sem),
          pltpu.async_copy(tieC, sh_outC.at[tieD], sem)]
    for h in hs:
        h.wait()
    plsc.subcore_barrier()

    hs = [pltpu.async_copy(sh_candU.at[pl.ds(cid * K_PAD, K_PAD)], dU, sem),
          pltpu.async_copy(sh_candI.at[pl.ds(cid * K_PAD, K_PAD)], dI, sem),
          pltpu.async_copy(sh_candC.at[pl.ds(cid * K_PAD, K_PAD)], dC, sem)]
    for h in hs:
        h.wait()
    dU[pl.ds(K_BG, 16)] = jnp.full((16,), INT_MIN, jnp.int32)
    dI[pl.ds(K_BG, 16)] = jnp.full((16,), INT_MAX, jnp.int32)

    def rank_block(bb, sta, stb, stc, std):
        ao = bb * 16
        aU = dU[pl.ds(ao, 16)]
        aI = dI[pl.ds(ao, 16)]
        aC = dC[pl.ds(ao, 16)]

        def b_body(j, acc):
            def r_body(r, acc):
                idx = j * 16 + ((iot + r) & 15)
                bU = plsc.load_gather(dU, [idx])
                bI = plsc.load_gather(dI, [idx])
                tie = jnp.where(bI < aI, one, zero)
                better = jnp.where(bU > aU, one,
                                   jnp.where(bU == aU, tie, zero))
                return acc + better

            return lax.fori_loop(0, 16, r_body, acc)

        rank = lax.fori_loop(0, K_PAD // 16, b_body, zero)
        lanepos = ao + iot
        wmask = jnp.where(lanepos < mvec, rank, kvec) < kvec
        rc = jnp.maximum(jnp.minimum(rank, kvec - 1), 0)
        ui = aU ^ sgn
        vbits = jnp.where(aU >= 0, aU, ~ui)
        sta[...] = plsc.bitcast(vbits, jnp.float32)
        stb[...] = aI
        stc[...] = aC
        std[...] = jnp.where(wmask, cid * OUT_PAD + rc, junk_o)
        hs = [pltpu.async_copy(sta, sh_outV.at[std], sem),
              pltpu.async_copy(stb, sh_outI.at[std], sem),
              pltpu.async_copy(stc, sh_outC.at[std], sem)]
        for h in hs:
            h.wait()

    rank_block(tid, st16a, st16b, st16c, st16d)

    @pl.when(tid < (K_PAD // 16) - 16)
    def _():
        rank_block(tid + 16, st16e, st16f, st16g, st16h)

    plsc.subcore_barrier()

    @pl.when((cid == 0) & (tid == 0))
    def _():
        hs = [pltpu.async_copy(sh_outV.at[pl.ds(0, K_FG)],
                               tieV.at[pl.ds(0, K_FG)], sem),
              pltpu.async_copy(sh_outI.at[pl.ds(0, K_FG)],
                               tieI.at[pl.ds(0, K_FG)], sem),
              pltpu.async_copy(sh_outC.at[pl.ds(0, K_FG)],
                               tieC.at[pl.ds(0, K_FG)], sem)]
        for h in hs:
            h.wait()
        hs = [pltpu.async_copy(tieV.at[pl.ds(0, K_FG)],
                               sv_out.at[pl.ds(0, K_FG)], sem),
              pltpu.async_copy(tieI.at[pl.ds(0, K_FG)],
                               si_out.at[pl.ds(0, K_FG)], sem),
              pltpu.async_copy(tieC.at[pl.ds(0, K_FG)],
                               scl_out.at[pl.ds(0, K_FG)], sem)]
        for h in hs:
            h.wait()

    @pl.when((cid == 1) & (tid == 0))
    def _():
        hs = [pltpu.async_copy(sh_outV.at[pl.ds(OUT_PAD, K_BG)],
                               tieV.at[pl.ds(0, K_BG)], sem),
              pltpu.async_copy(sh_outI.at[pl.ds(OUT_PAD, K_BG)],
                               tieI.at[pl.ds(0, K_BG)], sem),
              pltpu.async_copy(sh_outC.at[pl.ds(OUT_PAD, K_BG)],
                               tieC.at[pl.ds(0, K_BG)], sem)]
        for h in hs:
            h.wait()
        hs = [pltpu.async_copy(tieV.at[pl.ds(0, K_BG)],
                               sv_out.at[pl.ds(K_FG, K_BG)], sem),
              pltpu.async_copy(tieI.at[pl.ds(0, K_BG)],
                               si_out.at[pl.ds(K_FG, K_BG)], sem),
              pltpu.async_copy(tieC.at[pl.ds(0, K_BG)],
                               scl_out.at[pl.ds(K_FG, K_BG)], sem)]
        for h in hs:
            h.wait()


@jax.jit
def kernel(proposal_boxes, gt_boxes, scores, gt_classes):
    mesh = plsc.VectorSubcoreMesh(core_axis_name="c", subcore_axis_name="s",
                                  num_cores=2, num_subcores=16)
    f32 = jnp.float32
    i32 = jnp.int32

    pb = jnp.pad(proposal_boxes, ((0, N_PAD - N_PROP), (0, 0)))
    px1, py1, px2, py2 = (pb[:, i] for i in range(4))
    sc = jnp.pad(scores, (0, N_PAD - N_PROP))
    g_rep = [jnp.repeat(gt_boxes[:, i], 16) for i in range(4)]
    gcl_rep = jnp.repeat(gt_classes, 16)

    phase_a = pl.kernel(
        _phase_a_body,
        out_type=(
            jax.ShapeDtypeStruct((N_PAD,), f32),
            jax.ShapeDtypeStruct((N_PAD,), f32),
            jax.ShapeDtypeStruct((N_PAD,), f32),
            jax.ShapeDtypeStruct((N_PAD,), i32),
        ),
        mesh=mesh,
        compiler_params=pltpu.CompilerParams(needs_layout_passes=False),
        scratch_types=[
            pltpu.VMEM((CHUNK,), f32), pltpu.VMEM((CHUNK,), f32),
            pltpu.VMEM((CHUNK,), f32), pltpu.VMEM((CHUNK,), f32),
            pltpu.VMEM((CHUNK,), f32),
            pltpu.VMEM((N_GT * 16,), f32), pltpu.VMEM((N_GT * 16,), f32),
            pltpu.VMEM((N_GT * 16,), f32), pltpu.VMEM((N_GT * 16,), f32),
            pltpu.VMEM((N_GT * 16,), i32),
            pltpu.VMEM((CHUNK,), f32), pltpu.VMEM((CHUNK,), f32),
            pltpu.VMEM((CHUNK,), f32), pltpu.VMEM((CHUNK,), i32),
            pltpu.VMEM((N_GT * 16,), f32),
        ],
    )
    iou_p, fg_key, bg_key, cls = phase_a(
        px1, py1, px2, py2, sc, g_rep[0], g_rep[1], g_rep[2], g_rep[3], gcl_rep)

    phase_b = pl.kernel(
        _phase_b_body,
        out_type=(
            jax.ShapeDtypeStruct((512,), f32),
            jax.ShapeDtypeStruct((512,), i32),
            jax.ShapeDtypeStruct((512,), i32),
        ),
        mesh=mesh,
        compiler_params=pltpu.CompilerParams(needs_layout_passes=False),
        scratch_types=[
            pltpu.VMEM((SLICE,), f32),
            pltpu.VMEM((SLICE,), jnp.uint32),
            pltpu.VMEM((SLICE,), i32),
            pltpu.VMEM((2048,), i32),
            pltpu.VMEM((2048,), i32),
            pltpu.VMEM((K_PAD,), i32),
            pltpu.VMEM((K_PAD,), i32),
            pltpu.VMEM((K_PAD,), i32),
            pltpu.VMEM((K_PAD,), i32),
            pltpu.VMEM((K_PAD,), f32),
            pltpu.VMEM((K_PAD,), i32),
            pltpu.VMEM((K_PAD,), i32),
            pltpu.VMEM((K_PAD,), i32),
            pltpu.VMEM((K_PAD,), i32),
            pltpu.VMEM((K_PAD,), i32),
            pltpu.VMEM((K_PAD,), i32),
            pltpu.VMEM((16,), f32),
            pltpu.VMEM((16,), i32),
            pltpu.VMEM((16,), i32),
            pltpu.VMEM((16,), i32),
            pltpu.VMEM((16,), f32),
            pltpu.VMEM((16,), i32),
            pltpu.VMEM((16,), i32),
            pltpu.VMEM((16,), i32),
            pltpu.VMEM((32,), i32),
            pltpu.VMEM((512,), i32),
            pltpu.VMEM_SHARED((2 * 6144,), i32),
            pltpu.VMEM_SHARED((2 * 512,), i32),
            pltpu.VMEM_SHARED((2 * K_PAD,), i32),
            pltpu.VMEM_SHARED((2 * K_PAD,), i32),
            pltpu.VMEM_SHARED((2 * K_PAD,), i32),
            pltpu.VMEM_SHARED((2 * OUT_PAD,), f32),
            pltpu.VMEM_SHARED((2 * OUT_PAD,), i32),
            pltpu.VMEM_SHARED((2 * OUT_PAD,), i32),
            pltpu.SemaphoreType.DMA,
        ],
    )
    sv, si, scl = phase_b(fg_key, bg_key, cls)

    return iou_p[:N_PROP], si, scl, sv


# speedup vs baseline: 2.4886x; 1.0004x over previous
import jax
import jax.numpy as jnp
from jax import lax
from jax.experimental import pallas as pl
from jax.experimental.pallas import tpu as pltpu
from jax.experimental.pallas import tpu_sc as plsc

N_PROP = 20000
N_PAD = 20480
N_GT = 64
NW = 32
CHUNK = N_PAD // NW
NUM_CLASSES = 80
IOU_THRESH = 0.5
K_FG = 128
K_BG = 384
K_PAD = 400
OUT_PAD = 448
NEG_INF = float("-inf")


def _iota16():
    return lax.iota(jnp.int32, 16)


def _phase_a_body(px1, py1, px2, py2, sc, gx1, gy1, gx2, gy2, gcl,
                  iou_out, fg_out, bg_out, cls_out,
                  px1_v, py1_v, px2_v, py2_v, sc_v,
                  g1_v, g2_v, g3_v, g4_v, gc_v,
                  iou_s, fg_s, bg_s, cls_s, ag_v):
    wid = lax.axis_index("s") * 2 + lax.axis_index("c")
    base = wid * CHUNK

    pltpu.sync_copy(px1.at[pl.ds(base, CHUNK)], px1_v)
    pltpu.sync_copy(py1.at[pl.ds(base, CHUNK)], py1_v)
    pltpu.sync_copy(px2.at[pl.ds(base, CHUNK)], px2_v)
    pltpu.sync_copy(py2.at[pl.ds(base, CHUNK)], py2_v)
    pltpu.sync_copy(sc.at[pl.ds(base, CHUNK)], sc_v)
    pltpu.sync_copy(gx1, g1_v)
    pltpu.sync_copy(gy1, g2_v)
    pltpu.sync_copy(gx2, g3_v)
    pltpu.sync_copy(gy2, g4_v)
    pltpu.sync_copy(gcl, gc_v)

    def area_body(g, _):
        go = g * 16
        ag_v[pl.ds(go, 16)] = ((g3_v[pl.ds(go, 16)] - g1_v[pl.ds(go, 16)])
                               * (g4_v[pl.ds(go, 16)] - g2_v[pl.ds(go, 16)]))
        return ()

    lax.fori_loop(0, N_GT, area_body, ())

    UNR = 4

    def chunk_body(v, _):
        o0 = v * (16 * UNR)
        px1s = [px1_v[pl.ds(o0 + u * 16, 16)] for u in range(UNR)]
        py1s = [py1_v[pl.ds(o0 + u * 16, 16)] for u in range(UNR)]
        px2s = [px2_v[pl.ds(o0 + u * 16, 16)] for u in range(UNR)]
        py2s = [py2_v[pl.ds(o0 + u * 16, 16)] for u in range(UNR)]
        aps = [(px2s[u] - px1s[u]) * (py2s[u] - py1s[u]) for u in range(UNR)]

        def g_body(g, c):
            bis = list(c[:UNR])
            bcs = list(c[UNR:])
            go = g * 16
            g_x1 = g1_v[pl.ds(go, 16)]
            g_y1 = g2_v[pl.ds(go, 16)]
            g_x2 = g3_v[pl.ds(go, 16)]
            g_y2 = g4_v[pl.ds(go, 16)]
            g_c = gc_v[pl.ds(go, 16)]
            ag = ag_v[pl.ds(go, 16)]
            for u in range(UNR):
                w = jnp.maximum(
                    jnp.minimum(g_x2, px2s[u]) - jnp.maximum(g_x1, px1s[u]),
                    0.0)
                h = jnp.maximum(
                    jnp.minimum(g_y2, py2s[u]) - jnp.maximum(g_y1, py1s[u]),
                    0.0)
                inter = w * h
                iou = inter / jnp.maximum(ag + aps[u] - inter, 1e-9)
                upd = iou > bis[u]
                bis[u] = jnp.where(upd, iou, bis[u])
                bcs[u] = jnp.where(upd, g_c, bcs[u])
            return tuple(bis) + tuple(bcs)

        bi0 = jnp.full((16,), -1.0, jnp.float32)
        bc0 = jnp.zeros((16,), jnp.int32)
        res = lax.fori_loop(0, N_GT, g_body, (bi0,) * UNR + (bc0,) * UNR)

        neg = jnp.full((16,), NEG_INF, jnp.float32)
        for u in range(UNR):
            o, bi, bc = o0 + u * 16, res[u], res[UNR + u]
            s = sc_v[pl.ds(o, 16)]
            gidx = base + o + _iota16()
            valid = gidx < N_PROP
            matched = bi >= IOU_THRESH
            fg = jnp.where(
                valid,
                jnp.where(matched, bi, jnp.full((16,), -1.0, jnp.float32)),
                neg)
            bg = jnp.where(
                valid,
                jnp.where(matched, jnp.full((16,), -1e9, jnp.float32), s),
                neg)
            cl = jnp.where(matched, bc, jnp.full((16,), NUM_CLASSES,
                                                 jnp.int32))
            iou_s[pl.ds(o, 16)] = bi
            fg_s[pl.ds(o, 16)] = fg
            bg_s[pl.ds(o, 16)] = bg
            cls_s[pl.ds(o, 16)] = cl
        return ()

    lax.fori_loop(0, CHUNK // (16 * UNR), chunk_body, ())

    pltpu.sync_copy(iou_s, iou_out.at[pl.ds(base, CHUNK)])
    pltpu.sync_copy(fg_s, fg_out.at[pl.ds(base, CHUNK)])
    pltpu.sync_copy(bg_s, bg_out.at[pl.ds(base, CHUNK)])
    pltpu.sync_copy(cls_s, cls_out.at[pl.ds(base, CHUNK)])


SLICE = N_PAD // 16
NVB = SLICE // 16
INT_MIN = -2147483648
INT_MAX = 2147483647


def _phase_b_body(fg_hbm, bg_hbm, cls_hbm, sv_out, si_out, scl_out,
                  key_v, u_v, cls_v, hist, hidx,
                  candU, candI, candC, candD,
                  tieV, tieI, tieC, tieD,
                  dU, dI, dC, st16a, st16b, st16c, st16d,
                  st16e, st16f, st16g, st16h, cnt32, cntb,
                  sh_hist, sh_cnt, sh_candU, sh_candI, sh_candC,
                  sh_outV, sh_outI, sh_outC, sem):
    cid = lax.axis_index("c")
    tid = lax.axis_index("s")
    k = jnp.where(cid == 0, K_FG, K_BG)
    kvec = jnp.broadcast_to(k, (16,))
    one = jnp.ones((16,), jnp.int32)
    zero = jnp.zeros((16,), jnp.int32)
    iot = _iota16()
    sgn = jnp.int32(INT_MIN)
    base = tid * SLICE

    @pl.when(cid == 0)
    def _():
        h1 = pltpu.async_copy(fg_hbm.at[pl.ds(base, SLICE)], key_v, sem)
        h2 = pltpu.async_copy(cls_hbm.at[pl.ds(base, SLICE)], cls_v, sem)
        h1.wait()
        h2.wait()

    @pl.when(cid == 1)
    def _():
        h1 = pltpu.async_copy(bg_hbm.at[pl.ds(base, SLICE)], key_v, sem)
        h2 = pltpu.async_copy(cls_hbm.at[pl.ds(base, SLICE)], cls_v, sem)
        h1.wait()
        h2.wait()

    def z0_body(v, _):
        hist[pl.ds(v * 16, 16)] = zero
        return ()

    lax.fori_loop(0, 128, z0_body, ())
    hbase = cid * 6144
    hz = pltpu.async_copy(hist.at[pl.ds(0, 384)],
                          sh_hist.at[pl.ds(hbase + tid * 384, 384)], sem)

    def u_body(v, _):
        o = v * 16
        b = plsc.bitcast(key_v[pl.ds(o, 16)], jnp.int32)
        u = jnp.where(b < 0, ~b, b ^ sgn)
        u_v[pl.ds(o, 16)] = plsc.bitcast(u, jnp.uint32)
        return ()

    lax.fori_loop(0, NVB, u_body, ())
    hz.wait()
    plsc.subcore_barrier()

    pre = jnp.uint32(0)
    rem = k
    for pnum, (shift, width) in enumerate(((21, 11), (10, 11), (0, 10))):
        nbins = 1 << width
        nb = nbins // 16
        pbase = hbase + pnum * 2048

        def z_body(v, _, pbase=pbase):
            hidx[pl.ds(v * 16, 16)] = pbase + v * 16 + iot
            if pnum:
                hist[pl.ds(v * 16, 16)] = zero
            return ()

        lax.fori_loop(0, 128, z_body, ())

        dmask = jnp.uint32(nbins - 1)
        hi = shift + width
        pre_hi = lax.shift_right_logical(pre, jnp.uint32(hi)) if hi < 32 else None

        def acc_body(v, _, shift=shift, hi=hi, dmask=dmask, pre_hi=pre_hi):
            u = u_v[pl.ds(v * 16, 16)]
            dig = lax.shift_right_logical(u, jnp.uint32(shift)) & dmask
            if pre_hi is None:
                inc = one
            else:
                uh = lax.shift_right_logical(u, jnp.uint32(hi))
                inc = jnp.where(uh == jnp.broadcast_to(pre_hi, (16,)), one, zero)
            plsc.addupdate_scatter(hist, [plsc.bitcast(dig, jnp.int32)], inc)
            return ()

        lax.fori_loop(0, NVB, acc_body, ())

        pltpu.sync_copy(hist, sh_hist.at[hidx], add=True)
        plsc.subcore_barrier()
        pltpu.sync_copy(sh_hist.at[pl.ds(pbase, 2048)], hist.at[pl.ds(0, 2048)])

        def s_cond(c, nb=nb):
            j, above, found, d, g = c
            return (found == 0) & (j < nb)

        def s_step(c, nb=nb):
            j, above, found, d, g = c
            v = nb - 1 - j
            h = hist[pl.ds(v * 16, 16)]
            s_ge = lax.rev(plsc.cumsum(lax.rev(h, (0,))), (0,))
            tot = above + s_ge
            npos = jnp.max(
                jnp.where(tot >= jnp.broadcast_to(rem, (16,)), one, zero)
                * (iot + 1))
            lv = jnp.broadcast_to(npos - 1, (16,))
            gv = above + jnp.sum(jnp.where(iot > lv, h, zero))
            dv = v * 16 + npos - 1
            d = jnp.where(npos > 0, dv, d)
            g = jnp.where(npos > 0, gv, g)
            found = jnp.where(npos > 0, 1, 0)
            above = above + jnp.sum(h)
            return (j + 1, above, found, d, g)

        _, _, _, d, g = lax.while_loop(
            s_cond, s_step,
            (jnp.int32(0), jnp.int32(0), jnp.int32(0), jnp.int32(0),
             jnp.int32(0)))
        pre = pre | lax.shift_left(
            lax.bitcast_convert_type(d, jnp.uint32), jnp.uint32(shift))
        rem = rem - g

    t_u = pre
    tv = jnp.broadcast_to(t_u, (16,))

    def c_body(v, c):
        ng, ne = c
        u = u_v[pl.ds(v * 16, 16)]
        ng = ng + jnp.sum(jnp.where(u > tv, one, zero))
        ne = ne + jnp.sum(jnp.where(u == tv, one, zero))
        return (ng, ne)

    ng_me, ne_me = lax.fori_loop(0, NVB, c_body,
                                 (jnp.int32(0), jnp.int32(0)))
    cbase = cid * 512
    cnt32[pl.ds(0, 16)] = jnp.broadcast_to(ng_me, (16,))
    cnt32[pl.ds(16, 16)] = jnp.broadcast_to(ne_me, (16,))
    hc = pltpu.async_copy(cnt32, sh_cnt.at[pl.ds(cbase + tid * 32, 32)], sem)

    @pl.when(tid == 0)
    def _():
        def pad_body(v, _):
            o = v * 16
            dU[pl.ds(o, 16)] = jnp.full((16,), INT_MIN, jnp.int32)
            dI[pl.ds(o, 16)] = jnp.full((16,), INT_MAX, jnp.int32)
            return ()

        lax.fori_loop(0, K_PAD // 16, pad_body, ())
        p1 = pltpu.async_copy(dU, sh_candU.at[pl.ds(cid * K_PAD, K_PAD)], sem)
        p2 = pltpu.async_copy(dI, sh_candI.at[pl.ds(cid * K_PAD, K_PAD)], sem)
        p1.wait()
        p2.wait()

    hc.wait()
    plsc.subcore_barrier()

    pltpu.sync_copy(sh_cnt.at[pl.ds(cbase, 512)], cntb)
    g_cnt = plsc.load_gather(cntb, [iot * 33])
    e_cnt = plsc.load_gather(cntb, [iot * 33 + 16])
    m_tot = jnp.sum(g_cnt)
    ex_g = plsc.cumsum(g_cnt) - g_cnt
    ex_e = plsc.cumsum(e_cnt) - e_cnt
    tsel = jnp.where(iot == jnp.broadcast_to(tid, (16,)), one, zero)
    base_gt = jnp.sum(tsel * ex_g)
    base_eq = jnp.sum(tsel * ex_e)
    mvec = jnp.broadcast_to(m_tot, (16,))

    junk_c = jnp.broadcast_to(cid * K_PAD + K_BG + tid, (16,))
    junk_o = jnp.broadcast_to(cid * OUT_PAD + K_BG + tid, (16,))

    def pf_body(v, _):
        o = v * 16
        candD[pl.ds(o, 16)] = junk_c
        tieD[pl.ds(o, 16)] = junk_o
        return ()

    lax.fori_loop(0, K_PAD // 16, pf_body, ())

    def comp_body(v, carry):
        off, toff, eqc = carry
        o = v * 16
        u = u_v[pl.ds(o, 16)]
        any_rel = jnp.sum(jnp.where(u >= tv, one, zero))

        def do(carry):
            off, toff, eqc = carry
            kv = key_v[pl.ds(o, 16)]
            cv = cls_v[pl.ds(o, 16)]
            gidx = base + o + iot
            m_gt = u > tv
            s = plsc.bitcast(u, jnp.int32) ^ sgn
            dcand = cid * K_PAD + base_gt + off + plsc.cumsum(
                jnp.where(m_gt, one, zero)) - 1
            plsc.store_compressed(candU.at[pl.ds(off, 16)], s, mask=m_gt)
            plsc.store_compressed(candI.at[pl.ds(off, 16)], gidx, mask=m_gt)
            plsc.store_compressed(candC.at[pl.ds(off, 16)], cv, mask=m_gt)
            plsc.store_compressed(candD.at[pl.ds(off, 16)], dcand, mask=m_gt)
            n_gt = jnp.sum(jnp.where(m_gt, one, zero))
            m_eq = u == tv
            eq1 = jnp.where(m_eq, one, zero)
            pos = eqc + plsc.cumsum(eq1) - 1
            dest = mvec + jnp.broadcast_to(base_eq, (16,)) + pos
            keep = jnp.where(m_eq, dest, kvec) < kvec
            plsc.store_compressed(tieV.at[pl.ds(toff, 16)], kv, mask=keep)
            plsc.store_compressed(tieI.at[pl.ds(toff, 16)], gidx, mask=keep)
            plsc.store_compressed(tieC.at[pl.ds(toff, 16)], cv, mask=keep)
            plsc.store_compressed(tieD.at[pl.ds(toff, 16)],
                                  cid * OUT_PAD + dest, mask=keep)
            return (off + n_gt, toff + jnp.sum(jnp.where(keep, one, zero)),
                    eqc + jnp.sum(eq1))

        return lax.cond(any_rel > 0, do, lambda c: c, (off, toff, eqc))

    lax.fori_loop(0, NVB, comp_body,
                  (jnp.int32(0), jnp.int32(0), jnp.int32(0)))

    hs = [pltpu.async_copy(candU, sh_candU.at[candD], sem),
          pltpu.async_copy(candI, sh_candI.at[candD], sem),
          pltpu.async_copy(candC, sh_candC.at[candD], sem),
          pltpu.async_copy(tieV, sh_outV.at[tieD], sem),
          pltpu.async_copy(tieI, sh_outI.at[tieD], sem),
          pltpu.async_copy(tieC, sh_outC.at[tieD], sem)]
    for h in hs:
        h.wait()
    plsc.subcore_barrier()

    hs = [pltpu.async_copy(sh_candU.at[pl.ds(cid * K_PAD, K_PAD)], dU, sem),
          pltpu.async_copy(sh_candI.at[pl.ds(cid * K_PAD, K_PAD)], dI, sem),
          pltpu.async_copy(sh_candC.at[pl.ds(cid * K_PAD, K_PAD)], dC, sem)]
    for h in hs:
        h.wait()
    dU[pl.ds(K_BG, 16)] = jnp.full((16,), INT_MIN, jnp.int32)
    dI[pl.ds(K_BG, 16)] = jnp.full((16,), INT_MAX, jnp.int32)

    def rank_block(bb, sta, stb, stc, std):
        ao = bb * 16
        aU = dU[pl.ds(ao, 16)]
        aI = dI[pl.ds(ao, 16)]
        aC = dC[pl.ds(ao, 16)]

        def b_body(j, acc):
            def r_body(r, acc):
                idx = j * 16 + ((iot + r) & 15)
                bU = plsc.load_gather(dU, [idx])
                bI = plsc.load_gather(dI, [idx])
                tie = jnp.where(bI < aI, one, zero)
                better = jnp.where(bU > aU, one,
                                   jnp.where(bU == aU, tie, zero))
                return acc + better

            return lax.fori_loop(0, 16, r_body, acc)

        rank = lax.fori_loop(0, K_PAD // 16, b_body, zero)
        lanepos = ao + iot
        wmask = jnp.where(lanepos < mvec, rank, kvec) < kvec
        rc = jnp.maximum(jnp.minimum(rank, kvec - 1), 0)
        ui = aU ^ sgn
        vbits = jnp.where(aU >= 0, aU, ~ui)
        sta[...] = plsc.bitcast(vbits, jnp.float32)
        stb[...] = aI
        stc[...] = aC
        std[...] = jnp.where(wmask, cid * OUT_PAD + rc, junk_o)
        hs = [pltpu.async_copy(sta, sh_outV.at[std], sem),
              pltpu.async_copy(stb, sh_outI.at[std], sem),
              pltpu.async_copy(stc, sh_outC.at[std], sem)]
        for h in hs:
            h.wait()

    rank_block(tid, st16a, st16b, st16c, st16d)

    @pl.when(tid < (K_PAD // 16) - 16)
    def _():
        rank_block(tid + 16, st16e, st16f, st16g, st16h)

    plsc.subcore_barrier()

    @pl.when((cid == 0) & (tid == 0))
    def _():
        hs = [pltpu.async_copy(sh_outV.at[pl.ds(0, K_FG)],
                               tieV.at[pl.ds(0, K_FG)], sem),
              pltpu.async_copy(sh_outI.at[pl.ds(0, K_FG)],
                               tieI.at[pl.ds(0, K_FG)], sem),
              pltpu.async_copy(sh_outC.at[pl.ds(0, K_FG)],
                               tieC.at[pl.ds(0, K_FG)], sem)]
        for h in hs:
            h.wait()
        hs = [pltpu.async_copy(tieV.at[pl.ds(0, K_FG)],
                               sv_out.at[pl.ds(0, K_FG)], sem),
              pltpu.async_copy(tieI.at[pl.ds(0, K_FG)],
                               si_out.at[pl.ds(0, K_FG)], sem),
              pltpu.async_copy(tieC.at[pl.ds(0, K_FG)],
                               scl_out.at[pl.ds(0, K_FG)], sem)]
        for h in hs:
            h.wait()

    @pl.when((cid == 1) & (tid == 0))
    def _():
        hs = [pltpu.async_copy(sh_outV.at[pl.ds(OUT_PAD, K_BG)],
                               tieV.at[pl.ds(0, K_BG)], sem),
              pltpu.async_copy(sh_outI.at[pl.ds(OUT_PAD, K_BG)],
                               tieI.at[pl.ds(0, K_BG)], sem),
              pltpu.async_copy(sh_outC.at[pl.ds(OUT_PAD, K_BG)],
                               tieC.at[pl.ds(0, K_BG)], sem)]
        for h in hs:
            h.wait()
        hs = [pltpu.async_copy(tieV.at[pl.ds(0, K_BG)],
                               sv_out.at[pl.ds(K_FG, K_BG)], sem),
              pltpu.async_copy(tieI.at[pl.ds(0, K_BG)],
                               si_out.at[pl.ds(K_FG, K_BG)], sem),
              pltpu.async_copy(tieC.at[pl.ds(0, K_BG)],
                               scl_out.at[pl.ds(K_FG, K_BG)], sem)]
        for h in hs:
            h.wait()


@jax.jit
def kernel(proposal_boxes, gt_boxes, scores, gt_classes):
    mesh = plsc.VectorSubcoreMesh(core_axis_name="c", subcore_axis_name="s",
                                  num_cores=2, num_subcores=16)
    f32 = jnp.float32
    i32 = jnp.int32

    pb = jnp.pad(proposal_boxes, ((0, N_PAD - N_PROP), (0, 0)))
    px1, py1, px2, py2 = (pb[:, i] for i in range(4))
    sc = jnp.pad(scores, (0, N_PAD - N_PROP))
    g_rep = [jnp.repeat(gt_boxes[:, i], 16) for i in range(4)]
    gcl_rep = jnp.repeat(gt_classes, 16)

    phase_a = pl.kernel(
        _phase_a_body,
        out_type=(
            jax.ShapeDtypeStruct((N_PAD,), f32),
            jax.ShapeDtypeStruct((N_PAD,), f32),
            jax.ShapeDtypeStruct((N_PAD,), f32),
            jax.ShapeDtypeStruct((N_PAD,), i32),
        ),
        mesh=mesh,
        compiler_params=pltpu.CompilerParams(needs_layout_passes=False),
        scratch_types=[
            pltpu.VMEM((CHUNK,), f32), pltpu.VMEM((CHUNK,), f32),
            pltpu.VMEM((CHUNK,), f32), pltpu.VMEM((CHUNK,), f32),
            pltpu.VMEM((CHUNK,), f32),
            pltpu.VMEM((N_GT * 16,), f32), pltpu.VMEM((N_GT * 16,), f32),
            pltpu.VMEM((N_GT * 16,), f32), pltpu.VMEM((N_GT * 16,), f32),
            pltpu.VMEM((N_GT * 16,), i32),
            pltpu.VMEM((CHUNK,), f32), pltpu.VMEM((CHUNK,), f32),
            pltpu.VMEM((CHUNK,), f32), pltpu.VMEM((CHUNK,), i32),
            pltpu.VMEM((N_GT * 16,), f32),
        ],
    )
    iou_p, fg_key, bg_key, cls = phase_a(
        px1, py1, px2, py2, sc, g_rep[0], g_rep[1], g_rep[2], g_rep[3], gcl_rep)

    phase_b = pl.kernel(
        _phase_b_body,
        out_type=(
            jax.ShapeDtypeStruct((512,), f32),
            jax.ShapeDtypeStruct((512,), i32),
            jax.ShapeDtypeStruct((512,), i32),
        ),
        mesh=mesh,
        compiler_params=pltpu.CompilerParams(needs_layout_passes=False),
        scratch_types=[
            pltpu.VMEM((SLICE,), f32),
            pltpu.VMEM((SLICE,), jnp.uint32),
            pltpu.VMEM((SLICE,), i32),
            pltpu.VMEM((2048,), i32),
            pltpu.VMEM((2048,), i32),
            pltpu.VMEM((K_PAD,), i32),
            pltpu.VMEM((K_PAD,), i32),
            pltpu.VMEM((K_PAD,), i32),
            pltpu.VMEM((K_PAD,), i32),
            pltpu.VMEM((K_PAD,), f32),
            pltpu.VMEM((K_PAD,), i32),
            pltpu.VMEM((K_PAD,), i32),
            pltpu.VMEM((K_PAD,), i32),
            pltpu.VMEM((K_PAD,), i32),
            pltpu.VMEM((K_PAD,), i32),
            pltpu.VMEM((K_PAD,), i32),
            pltpu.VMEM((16,), f32),
            pltpu.VMEM((16,), i32),
            pltpu.VMEM((16,), i32),
            pltpu.VMEM((16,), i32),
            pltpu.VMEM((16,), f32),
            pltpu.VMEM((16,), i32),
            pltpu.VMEM((16,), i32),
            pltpu.VMEM((16,), i32),
            pltpu.VMEM((32,), i32),
            pltpu.VMEM((512,), i32),
            pltpu.VMEM_SHARED((2 * 6144,), i32),
            pltpu.VMEM_SHARED((2 * 512,), i32),
            pltpu.VMEM_SHARED((2 * K_PAD,), i32),
            pltpu.VMEM_SHARED((2 * K_PAD,), i32),
            pltpu.VMEM_SHARED((2 * K_PAD,), i32),
            pltpu.VMEM_SHARED((2 * OUT_PAD,), f32),
            pltpu.VMEM_SHARED((2 * OUT_PAD,), i32),
            pltpu.VMEM_SHARED((2 * OUT_PAD,), i32),
            pltpu.SemaphoreType.DMA,
        ],
    )
    sv, si, scl = phase_b(fg_key, bg_key, cls)

    return iou_p[:N_PROP], si, scl, sv
